# Initial kernel scaffold; baseline (speedup 1.0000x reference)
import jax
import jax.numpy as jnp
from jax.experimental import pallas as pl


def kernel(x, edge_index, W1, b1, W2, b2):
    raise NotImplementedError("write your pallas kernel here")


# speedup vs baseline: 9.1157x; 9.1157x over previous
import functools

import jax
import jax.numpy as jnp
from jax import lax
from jax.experimental import pallas as pl
from jax.experimental.pallas import tpu as pltpu
from jax.experimental.pallas import tpu_sc as plsc

N = 10000
F_IN = 128
HID = 64
CLS = 32

NC = 2
NS = 16
L = 16
NW = NC * NS

CH = 128
N_ACC = 10240
R = N_ACC // NS
DUMMY = N

BLK = 400
GRID = N // BLK

_sc_mesh = functools.partial(
    plsc.VectorSubcoreMesh, core_axis_name="c", subcore_axis_name="s")


def _make_deg(nch):
    @functools.partial(
        pl.kernel,
        out_type=jax.ShapeDtypeStruct((NC, N_ACC), jnp.float32),
        mesh=_sc_mesh(),
        compiler_params=pltpu.CompilerParams(needs_layout_passes=False, use_tc_tiling_on_sc=False),
        scratch_types=[
            pltpu.VMEM((nch, CH), jnp.int32),
            pltpu.VMEM((nch, CH), jnp.int32),
            pltpu.VMEM((N_ACC,), jnp.float32),
            pltpu.VMEM((R,), jnp.float32),
            pltpu.VMEM((R,), jnp.float32),
            pltpu.VMEM_SHARED((NS, N_ACC), jnp.float32),
        ],
    )
    def deg(row_hbm, col_hbm, out_hbm, rows_v, cols_v, hist, accv, tmpv, sh):
        c = lax.axis_index("c")
        s = lax.axis_index("s")
        wid = c * NS + s
        zero16 = jnp.zeros((L,), jnp.float32)

        def z(i, _):
            hist[pl.ds(i * L, L)] = zero16
            return 0
        lax.fori_loop(0, N_ACC // L, z, 0)

        pltpu.sync_copy(row_hbm.at[pl.ds(wid * nch, nch)], rows_v)
        pltpu.sync_copy(col_hbm.at[pl.ds(wid * nch, nch)], cols_v)

        ones16 = jnp.ones((L,), jnp.float32)

        def count(j, _):
            for k in range(CH // L):
                r = rows_v[j, pl.ds(k * L, L)]
                cc = cols_v[j, pl.ds(k * L, L)]
                plsc.addupdate_scatter(hist, [r], ones16, mask=r != cc)
            return 0
        lax.fori_loop(0, nch, count, 0)

        pltpu.sync_copy(hist, sh.at[s])
        plsc.subcore_barrier()

        pltpu.sync_copy(sh.at[0, pl.ds(s * R, R)], accv)
        for t in range(1, NS):
            pltpu.sync_copy(sh.at[t, pl.ds(s * R, R)], tmpv)

            def addl(i, _):
                accv[pl.ds(i * L, L)] = (accv[pl.ds(i * L, L)]
                                         + tmpv[pl.ds(i * L, L)])
                return 0
            lax.fori_loop(0, R // L, addl, 0)
        pltpu.sync_copy(accv, out_hbm.at[c, pl.ds(s * R, R)])

    return deg


def _make_prop(d, nch):
    @functools.partial(
        pl.kernel,
        out_type=jax.ShapeDtypeStruct((NC, N_ACC, d), jnp.float32),
        mesh=_sc_mesh(),
        compiler_params=pltpu.CompilerParams(needs_layout_passes=False, use_tc_tiling_on_sc=False),
        scratch_types=[
            pltpu.VMEM((nch, CH), jnp.int32),
            pltpu.VMEM((nch, CH), jnp.int32),
            pltpu.VMEM((CH, d), jnp.float32),
            pltpu.VMEM((CH, d), jnp.float32),
            pltpu.SemaphoreType.DMA,
            pltpu.SemaphoreType.DMA,
            pltpu.VMEM_SHARED((N_ACC, d), jnp.float32),
        ],
    )
    def prop(u_hbm, row_hbm, col_hbm, out_hbm,
             rows_v, cols_v, gb0, gb1, sem0, sem1, acc):
        c = lax.axis_index("c")
        s = lax.axis_index("s")
        wid = c * NS + s
        zero16 = jnp.zeros((L,), jnp.float32)

        def zrow(i, _):
            for k in range(d // L):
                gb0[i, pl.ds(k * L, L)] = zero16
            return 0
        lax.fori_loop(0, CH, zrow, 0)
        for t in range(R // CH):
            pltpu.sync_copy(gb0, acc.at[pl.ds(s * R + t * CH, CH)])

        pltpu.sync_copy(row_hbm.at[pl.ds(wid * nch, nch)], rows_v)
        pltpu.sync_copy(col_hbm.at[pl.ds(wid * nch, nch)], cols_v)

        def fix(j, _):
            for k in range(CH // L):
                r = rows_v[j, pl.ds(k * L, L)]
                cc = cols_v[j, pl.ds(k * L, L)]
                cols_v[j, pl.ds(k * L, L)] = jnp.where(r == cc, DUMMY, cc)
            return 0
        lax.fori_loop(0, nch, fix, 0)

        plsc.subcore_barrier()

        def pair(p, _):
            j0 = p * 2
            j1 = j0 + 1
            pltpu.async_copy(u_hbm.at[rows_v.at[j0]], gb0, sem0)
            pltpu.async_copy(u_hbm.at[rows_v.at[j1]], gb1, sem1)
            pltpu.make_async_copy(u_hbm.at[rows_v.at[j0]], gb0, sem0).wait()
            pltpu.sync_copy(gb0, acc.at[cols_v.at[j0]], add=True)
            pltpu.make_async_copy(u_hbm.at[rows_v.at[j1]], gb1, sem1).wait()
            pltpu.sync_copy(gb1, acc.at[cols_v.at[j1]], add=True)
            return 0
        lax.fori_loop(0, nch // 2, pair, 0)

        plsc.subcore_barrier()
        pltpu.sync_copy(acc.at[pl.ds(s * R, R)],
                        out_hbm.at[c, pl.ds(s * R, R)])

    return prop


def _row_spec(d):
    return pl.BlockSpec((BLK, d), lambda i: (i, 0))


def _parts_spec(d):
    return pl.BlockSpec((NC, BLK, d), lambda i: (0, i, 0))


def _full_spec(shape):
    nd = len(shape)
    return pl.BlockSpec(shape, lambda i, _nd=nd: (0,) * _nd)


def _tc1(x, w1, parts):
    def body(x_ref, w_ref, p_ref, xw0_ref, xw1_ref, u1_ref, dinv_ref):
        degb = p_ref[0] + p_ref[1]
        dinv = jnp.where(degb > 0,
                         lax.rsqrt(jnp.where(degb > 0, degb, 1.0)), 0.0)
        xb = x_ref[...]
        w = w_ref[...]
        xw0_ref[...] = jnp.dot(xb, w[0] - w[2],
                               preferred_element_type=jnp.float32)
        xw1_ref[...] = jnp.dot(xb, w[1], preferred_element_type=jnp.float32)
        u1_ref[...] = dinv * jnp.dot(xb, w[2],
                                     preferred_element_type=jnp.float32)
        dinv_ref[...] = dinv

    o = jax.ShapeDtypeStruct((N, HID), jnp.float32)
    return pl.pallas_call(
        body,
        grid=(GRID,),
        in_specs=[_row_spec(F_IN), _full_spec((3, F_IN, HID)), _parts_spec(1)],
        out_specs=[_row_spec(HID), _row_spec(HID), _row_spec(HID),
                   _row_spec(1)],
        out_shape=[o, o, o, jax.ShapeDtypeStruct((N, 1), jnp.float32)],
    )(x, w1, parts)


def _tc_mid(d):
    def body(a_ref, p_ref, dinv_ref, v_ref):
        g = p_ref[0] + p_ref[1]
        dinv = dinv_ref[...]
        v_ref[...] = dinv * (a_ref[...] - 2.0 * dinv * g)

    def run(a, parts, dinv):
        return pl.pallas_call(
            body,
            grid=(GRID,),
            in_specs=[_row_spec(d), _parts_spec(d), _row_spec(1)],
            out_specs=_row_spec(d),
            out_shape=jax.ShapeDtypeStruct((N, d), jnp.float32),
        )(a, parts, dinv)
    return run


def _tc3(xw0, parts, dinv, b1, w2):
    def body(xw0_ref, p_ref, dinv_ref, b_ref, w_ref,
             y0_ref, y1_ref, u2_ref):
        g = p_ref[0] + p_ref[1]
        dinv = dinv_ref[...]
        h = jax.nn.relu(xw0_ref[...] - dinv * g + b_ref[...])
        w = w_ref[...]
        y0_ref[...] = jnp.dot(h, w[0] - w[2],
                              preferred_element_type=jnp.float32)
        y1_ref[...] = jnp.dot(h, w[1], preferred_element_type=jnp.float32)
        u2_ref[...] = dinv * jnp.dot(h, w[2],
                                     preferred_element_type=jnp.float32)

    o = jax.ShapeDtypeStruct((N, CLS), jnp.float32)
    return pl.pallas_call(
        body,
        grid=(GRID,),
        in_specs=[_row_spec(HID), _parts_spec(HID), _row_spec(1),
                  _full_spec((1, HID)), _full_spec((3, HID, CLS))],
        out_specs=[_row_spec(CLS), _row_spec(CLS), _row_spec(CLS)],
        out_shape=[o, o, o],
    )(xw0, parts, dinv, b1, w2)


def _tc5(y0, parts, dinv, b2):
    def body(y0_ref, p_ref, dinv_ref, b_ref, out_ref):
        g = p_ref[0] + p_ref[1]
        z = y0_ref[...] - dinv_ref[...] * g + b_ref[...]
        m = jnp.max(z, axis=1, keepdims=True)
        e = jnp.exp(z - m)
        out_ref[...] = (z - m) - jnp.log(jnp.sum(e, axis=1, keepdims=True))

    return pl.pallas_call(
        body,
        grid=(GRID,),
        in_specs=[_row_spec(CLS), _parts_spec(CLS), _row_spec(1),
                  _full_spec((1, CLS))],
        out_specs=_row_spec(CLS),
        out_shape=jax.ShapeDtypeStruct((N, CLS), jnp.float32),
    )(y0, parts, dinv, b2)


def kernel(x, edge_index, W1, b1, W2, b2):
    e = edge_index.shape[1]
    nch = -(-e // (NW * CH))
    nch += nch % 2
    e_pad = NW * CH * nch
    row = jnp.pad(edge_index[0], (0, e_pad - e)).reshape(e_pad // CH, CH)
    col = jnp.pad(edge_index[1], (0, e_pad - e)).reshape(e_pad // CH, CH)

    degp = _make_deg(nch)(row, col).reshape(NC, N_ACC, 1)
    xw0, xw1, u1, dinv = _tc1(x, W1, degp)

    prop64 = _make_prop(HID, nch)
    prop32 = _make_prop(CLS, nch)

    g1 = prop64(u1, row, col)
    v1 = _tc_mid(HID)(xw1, g1, dinv)
    g2 = prop64(v1, row, col)
    y0, y1, u2 = _tc3(xw0, g2, dinv, b1.reshape(1, HID), W2)
    g3 = prop32(u2, row, col)
    v2 = _tc_mid(CLS)(y1, g3, dinv)
    g4 = prop32(v2, row, col)
    return _tc5(y0, g4, dinv, b2.reshape(1, CLS))


# speedup vs baseline: 9.9874x; 1.0956x over previous
import functools

import jax
import jax.numpy as jnp
from jax import lax
from jax.experimental import pallas as pl
from jax.experimental.pallas import tpu as pltpu
from jax.experimental.pallas import tpu_sc as plsc

N = 10000
F_IN = 128
HID = 64
CLS = 32

NC = 2
NS = 16
L = 16
NW = NC * NS

CH = 128
N_ACC = 10240
R = N_ACC // NS
DUMMY = N

NBUF = 4

BLK = 400
GRID = N // BLK

_sc_mesh = functools.partial(
    plsc.VectorSubcoreMesh, core_axis_name="c", subcore_axis_name="s")


def _make_deg(nch):
    @functools.partial(
        pl.kernel,
        out_type=jax.ShapeDtypeStruct((NC, N_ACC), jnp.float32),
        mesh=_sc_mesh(),
        compiler_params=pltpu.CompilerParams(needs_layout_passes=False, use_tc_tiling_on_sc=False),
        scratch_types=[
            pltpu.VMEM((nch, CH), jnp.int32),
            pltpu.VMEM((nch, CH), jnp.int32),
            pltpu.VMEM((N_ACC,), jnp.float32),
            pltpu.VMEM((R,), jnp.float32),
            pltpu.VMEM((R,), jnp.float32),
            pltpu.VMEM_SHARED((NS, N_ACC), jnp.float32),
        ],
    )
    def deg(row_hbm, col_hbm, out_hbm, rows_v, cols_v, hist, accv, tmpv, sh):
        c = lax.axis_index("c")
        s = lax.axis_index("s")
        wid = c * NS + s
        zero16 = jnp.zeros((L,), jnp.float32)

        def z(i, _):
            hist[pl.ds(i * L, L)] = zero16
            return 0
        lax.fori_loop(0, N_ACC // L, z, 0)

        pltpu.sync_copy(row_hbm.at[pl.ds(wid * nch, nch)], rows_v)
        pltpu.sync_copy(col_hbm.at[pl.ds(wid * nch, nch)], cols_v)

        ones16 = jnp.ones((L,), jnp.float32)

        def count(j, _):
            for k in range(CH // L):
                r = rows_v[j, pl.ds(k * L, L)]
                cc = cols_v[j, pl.ds(k * L, L)]
                plsc.addupdate_scatter(hist, [r], ones16, mask=r != cc)
            return 0
        lax.fori_loop(0, nch, count, 0)

        pltpu.sync_copy(hist, sh.at[s])
        plsc.subcore_barrier()

        pltpu.sync_copy(sh.at[0, pl.ds(s * R, R)], accv)
        for t in range(1, NS):
            pltpu.sync_copy(sh.at[t, pl.ds(s * R, R)], tmpv)

            def addl(i, _):
                accv[pl.ds(i * L, L)] = (accv[pl.ds(i * L, L)]
                                         + tmpv[pl.ds(i * L, L)])
                return 0
            lax.fori_loop(0, R // L, addl, 0)
        pltpu.sync_copy(accv, out_hbm.at[c, pl.ds(s * R, R)])

    return deg


def _make_prop(d, nch):
    @functools.partial(
        pl.kernel,
        out_type=jax.ShapeDtypeStruct((NC, N_ACC, d), jnp.float32),
        mesh=_sc_mesh(),
        compiler_params=pltpu.CompilerParams(needs_layout_passes=False, use_tc_tiling_on_sc=False),
        scratch_types=[
            pltpu.VMEM((nch, CH), jnp.int32),
            pltpu.VMEM((nch, CH), jnp.int32),
            [pltpu.VMEM((CH, d), jnp.float32) for _ in range(NBUF)],
            [pltpu.SemaphoreType.DMA for _ in range(NBUF)],
            [pltpu.SemaphoreType.DMA for _ in range(NBUF)],
            pltpu.VMEM_SHARED((N_ACC, d), jnp.float32),
        ],
    )
    def prop(u_hbm, row_hbm, col_hbm, out_hbm,
             rows_v, cols_v, gb, sem_g, sem_s, acc):
        c = lax.axis_index("c")
        s = lax.axis_index("s")
        wid = c * NS + s
        zero16 = jnp.zeros((L,), jnp.float32)

        def zrow(i, _):
            for k in range(d // L):
                gb[0][i, pl.ds(k * L, L)] = zero16
            return 0
        lax.fori_loop(0, CH, zrow, 0)
        for t in range(R // CH):
            pltpu.sync_copy(gb[0], acc.at[pl.ds(s * R + t * CH, CH)])

        pltpu.sync_copy(row_hbm.at[pl.ds(wid * nch, nch)], rows_v)
        pltpu.sync_copy(col_hbm.at[pl.ds(wid * nch, nch)], cols_v)

        def fix(j, _):
            for k in range(CH // L):
                r = rows_v[j, pl.ds(k * L, L)]
                cc = cols_v[j, pl.ds(k * L, L)]
                cols_v[j, pl.ds(k * L, L)] = jnp.where(r == cc, DUMMY, cc)
            return 0
        lax.fori_loop(0, nch, fix, 0)

        plsc.subcore_barrier()

        def group(p, _):
            j0 = p * NBUF
            for b in range(NBUF):
                jb = j0 + b

                @pl.when(p > 0)
                def _():
                    pltpu.make_async_copy(
                        gb[b], acc.at[cols_v.at[jb]], sem_s[b]).wait()
                pltpu.async_copy(u_hbm.at[rows_v.at[jb]], gb[b], sem_g[b])
            for b in range(NBUF):
                jb = j0 + b
                pltpu.make_async_copy(
                    u_hbm.at[rows_v.at[jb]], gb[b], sem_g[b]).wait()
                pltpu.async_copy(
                    gb[b], acc.at[cols_v.at[jb]], sem_s[b], add=True)
            return 0
        lax.fori_loop(0, nch // NBUF, group, 0)
        for b in range(NBUF):
            pltpu.make_async_copy(gb[b], acc.at[cols_v.at[b]], sem_s[b]).wait()

        plsc.subcore_barrier()
        pltpu.sync_copy(acc.at[pl.ds(s * R, R)],
                        out_hbm.at[c, pl.ds(s * R, R)])

    return prop


def _row_spec(d):
    return pl.BlockSpec((BLK, d), lambda i: (i, 0))


def _parts_spec(d):
    return pl.BlockSpec((NC, BLK, d), lambda i: (0, i, 0))


def _full_spec(shape):
    nd = len(shape)
    return pl.BlockSpec(shape, lambda i, _nd=nd: (0,) * _nd)


def _tc1(x, w1, parts):
    def body(x_ref, w_ref, p_ref, xw0_ref, xw1_ref, u1_ref, dinv_ref):
        degb = p_ref[0] + p_ref[1]
        dinv = jnp.where(degb > 0,
                         lax.rsqrt(jnp.where(degb > 0, degb, 1.0)), 0.0)
        xb = x_ref[...]
        w = w_ref[...]
        xw0_ref[...] = jnp.dot(xb, w[0] - w[2],
                               preferred_element_type=jnp.float32)
        xw1_ref[...] = jnp.dot(xb, w[1], preferred_element_type=jnp.float32)
        u1_ref[...] = dinv * jnp.dot(xb, w[2],
                                     preferred_element_type=jnp.float32)
        dinv_ref[...] = dinv

    o = jax.ShapeDtypeStruct((N, HID), jnp.float32)
    return pl.pallas_call(
        body,
        grid=(GRID,),
        in_specs=[_row_spec(F_IN), _full_spec((3, F_IN, HID)), _parts_spec(1)],
        out_specs=[_row_spec(HID), _row_spec(HID), _row_spec(HID),
                   _row_spec(1)],
        out_shape=[o, o, o, jax.ShapeDtypeStruct((N, 1), jnp.float32)],
    )(x, w1, parts)


def _tc_mid(d):
    def body(a_ref, p_ref, dinv_ref, v_ref):
        g = p_ref[0] + p_ref[1]
        dinv = dinv_ref[...]
        v_ref[...] = dinv * (a_ref[...] - 2.0 * dinv * g)

    def run(a, parts, dinv):
        return pl.pallas_call(
            body,
            grid=(GRID,),
            in_specs=[_row_spec(d), _parts_spec(d), _row_spec(1)],
            out_specs=_row_spec(d),
            out_shape=jax.ShapeDtypeStruct((N, d), jnp.float32),
        )(a, parts, dinv)
    return run


def _tc3(xw0, parts, dinv, b1, w2):
    def body(xw0_ref, p_ref, dinv_ref, b_ref, w_ref,
             y0_ref, y1_ref, u2_ref):
        g = p_ref[0] + p_ref[1]
        dinv = dinv_ref[...]
        h = jax.nn.relu(xw0_ref[...] - dinv * g + b_ref[...])
        w = w_ref[...]
        y0_ref[...] = jnp.dot(h, w[0] - w[2],
                              preferred_element_type=jnp.float32)
        y1_ref[...] = jnp.dot(h, w[1], preferred_element_type=jnp.float32)
        u2_ref[...] = dinv * jnp.dot(h, w[2],
                                     preferred_element_type=jnp.float32)

    o = jax.ShapeDtypeStruct((N, CLS), jnp.float32)
    return pl.pallas_call(
        body,
        grid=(GRID,),
        in_specs=[_row_spec(HID), _parts_spec(HID), _row_spec(1),
                  _full_spec((1, HID)), _full_spec((3, HID, CLS))],
        out_specs=[_row_spec(CLS), _row_spec(CLS), _row_spec(CLS)],
        out_shape=[o, o, o],
    )(xw0, parts, dinv, b1, w2)


def _tc5(y0, parts, dinv, b2):
    def body(y0_ref, p_ref, dinv_ref, b_ref, out_ref):
        g = p_ref[0] + p_ref[1]
        z = y0_ref[...] - dinv_ref[...] * g + b_ref[...]
        m = jnp.max(z, axis=1, keepdims=True)
        e = jnp.exp(z - m)
        out_ref[...] = (z - m) - jnp.log(jnp.sum(e, axis=1, keepdims=True))

    return pl.pallas_call(
        body,
        grid=(GRID,),
        in_specs=[_row_spec(CLS), _parts_spec(CLS), _row_spec(1),
                  _full_spec((1, CLS))],
        out_specs=_row_spec(CLS),
        out_shape=jax.ShapeDtypeStruct((N, CLS), jnp.float32),
    )(y0, parts, dinv, b2)


def kernel(x, edge_index, W1, b1, W2, b2):
    e = edge_index.shape[1]
    nch = -(-e // (NW * CH))
    nch = -(-nch // NBUF) * NBUF
    e_pad = NW * CH * nch
    row = jnp.pad(edge_index[0], (0, e_pad - e)).reshape(e_pad // CH, CH)
    col = jnp.pad(edge_index[1], (0, e_pad - e)).reshape(e_pad // CH, CH)

    degp = _make_deg(nch)(row, col).reshape(NC, N_ACC, 1)
    xw0, xw1, u1, dinv = _tc1(x, W1, degp)

    prop64 = _make_prop(HID, nch)
    prop32 = _make_prop(CLS, nch)

    g1 = prop64(u1, row, col)
    v1 = _tc_mid(HID)(xw1, g1, dinv)
    g2 = prop64(v1, row, col)
    y0, y1, u2 = _tc3(xw0, g2, dinv, b1.reshape(1, HID), W2)
    g3 = prop32(u2, row, col)
    v2 = _tc_mid(CLS)(y1, g3, dinv)
    g4 = prop32(v2, row, col)
    return _tc5(y0, g4, dinv, b2.reshape(1, CLS))


# speedup vs baseline: 15.1748x; 1.5194x over previous
import functools

import jax
import jax.numpy as jnp
from jax import lax
from jax.experimental import pallas as pl
from jax.experimental.pallas import tpu as pltpu
from jax.experimental.pallas import tpu_sc as plsc

N = 10000
F_IN = 128
HID = 64
CLS = 32

NC = 2
NS = 16
L = 16
NW = NC * NS

CH = 128
N_ACC = 10240
R = N_ACC // NS
DUMMY = N

NBUF = 4

BLK = 400
GRID = N // BLK

_sc_mesh = functools.partial(
    plsc.VectorSubcoreMesh, core_axis_name="c", subcore_axis_name="s")


def _make_deg(nch):
    @functools.partial(
        pl.kernel,
        out_type=jax.ShapeDtypeStruct((NC, N_ACC), jnp.float32),
        mesh=_sc_mesh(),
        compiler_params=pltpu.CompilerParams(needs_layout_passes=False, use_tc_tiling_on_sc=False),
        scratch_types=[
            pltpu.VMEM((nch, CH), jnp.int32),
            pltpu.VMEM((nch, CH), jnp.int32),
            pltpu.VMEM((N_ACC,), jnp.float32),
            pltpu.VMEM((R,), jnp.float32),
            pltpu.VMEM((R,), jnp.float32),
            pltpu.VMEM_SHARED((NS, N_ACC), jnp.float32),
        ],
    )
    def deg(row_hbm, col_hbm, out_hbm, rows_v, cols_v, hist, accv, tmpv, sh):
        c = lax.axis_index("c")
        s = lax.axis_index("s")
        wid = c * NS + s
        zero16 = jnp.zeros((L,), jnp.float32)

        def z(i, _):
            hist[pl.ds(i * L, L)] = zero16
            return 0
        lax.fori_loop(0, N_ACC // L, z, 0)

        pltpu.sync_copy(row_hbm.at[pl.ds(wid * nch, nch)], rows_v)
        pltpu.sync_copy(col_hbm.at[pl.ds(wid * nch, nch)], cols_v)

        ones16 = jnp.ones((L,), jnp.float32)

        def count(j, _):
            for k in range(CH // L):
                r = rows_v[j, pl.ds(k * L, L)]
                cc = cols_v[j, pl.ds(k * L, L)]
                plsc.addupdate_scatter(hist, [r], ones16, mask=r != cc)
            return 0
        lax.fori_loop(0, nch, count, 0)

        pltpu.sync_copy(hist, sh.at[s])
        plsc.subcore_barrier()

        pltpu.sync_copy(sh.at[0, pl.ds(s * R, R)], accv)
        for t in range(1, NS):
            pltpu.sync_copy(sh.at[t, pl.ds(s * R, R)], tmpv)

            def addl(i, _):
                accv[pl.ds(i * L, L)] = (accv[pl.ds(i * L, L)]
                                         + tmpv[pl.ds(i * L, L)])
                return 0
            lax.fori_loop(0, R // L, addl, 0)
        pltpu.sync_copy(accv, out_hbm.at[c, pl.ds(s * R, R)])

    return deg


def _make_prop(d, nch2):
    hd = d // NC

    @functools.partial(
        pl.kernel,
        out_type=jax.ShapeDtypeStruct((N_ACC, NC, hd), jnp.float32),
        mesh=_sc_mesh(),
        compiler_params=pltpu.CompilerParams(needs_layout_passes=False, use_tc_tiling_on_sc=False),
        scratch_types=[
            pltpu.VMEM((nch2, CH), jnp.int32),
            pltpu.VMEM((nch2, CH), jnp.int32),
            [pltpu.VMEM((CH, hd), jnp.float32) for _ in range(NBUF)],
            [pltpu.SemaphoreType.DMA for _ in range(NBUF)],
            [pltpu.SemaphoreType.DMA for _ in range(NBUF)],
            pltpu.VMEM_SHARED((N_ACC, hd), jnp.float32),
            pltpu.VMEM_SHARED((N, hd), jnp.float32),
        ],
    )
    def prop(u_hbm, row_hbm, col_hbm, out_hbm,
             rows_v, cols_v, gb, sem_g, sem_s, acc, u_sh):
        c = lax.axis_index("c")
        s = lax.axis_index("s")
        zero16 = jnp.zeros((L,), jnp.float32)

        nrs = N // NS
        pltpu.sync_copy(u_hbm.at[pl.ds(s * nrs, nrs), pl.ds(c * hd, hd)],
                        u_sh.at[pl.ds(s * nrs, nrs)])

        def zrow(i, _):
            for k in range(hd // L):
                gb[0][i, pl.ds(k * L, L)] = zero16
            return 0
        lax.fori_loop(0, CH, zrow, 0)
        for t in range(R // CH):
            pltpu.sync_copy(gb[0], acc.at[pl.ds(s * R + t * CH, CH)])

        pltpu.sync_copy(row_hbm.at[pl.ds(s * nch2, nch2)], rows_v)
        pltpu.sync_copy(col_hbm.at[pl.ds(s * nch2, nch2)], cols_v)

        def fix(j, _):
            for k in range(CH // L):
                r = rows_v[j, pl.ds(k * L, L)]
                cc = cols_v[j, pl.ds(k * L, L)]
                cols_v[j, pl.ds(k * L, L)] = jnp.where(r == cc, DUMMY, cc)
            return 0
        lax.fori_loop(0, nch2, fix, 0)

        plsc.subcore_barrier()

        def group(p, _):
            j0 = p * NBUF
            for b in range(NBUF):
                jb = j0 + b

                @pl.when(p > 0)
                def _():
                    pltpu.make_async_copy(
                        gb[b], acc.at[cols_v.at[jb]], sem_s[b]).wait()
                pltpu.async_copy(u_sh.at[rows_v.at[jb]], gb[b], sem_g[b])
            for b in range(NBUF):
                jb = j0 + b
                pltpu.make_async_copy(
                    u_sh.at[rows_v.at[jb]], gb[b], sem_g[b]).wait()
                pltpu.async_copy(
                    gb[b], acc.at[cols_v.at[jb]], sem_s[b], add=True)
            return 0
        lax.fori_loop(0, nch2 // NBUF, group, 0)
        for b in range(NBUF):
            pltpu.make_async_copy(gb[b], acc.at[cols_v.at[b]], sem_s[b]).wait()

        plsc.subcore_barrier()
        pltpu.sync_copy(acc.at[pl.ds(s * R, R)],
                        out_hbm.at[pl.ds(s * R, R), c])

    return prop


def _row_spec(d):
    return pl.BlockSpec((BLK, d), lambda i: (i, 0))


def _parts_spec(d):
    return pl.BlockSpec((NC, BLK, d), lambda i: (0, i, 0))


def _full_spec(shape):
    nd = len(shape)
    return pl.BlockSpec(shape, lambda i, _nd=nd: (0,) * _nd)


def _tc1(x, w1, parts):
    def body(x_ref, w_ref, p_ref, xw0_ref, xw1_ref, u1_ref, dinv_ref):
        degb = p_ref[0] + p_ref[1]
        dinv = jnp.where(degb > 0,
                         lax.rsqrt(jnp.where(degb > 0, degb, 1.0)), 0.0)
        xb = x_ref[...]
        w = w_ref[...]
        xw0_ref[...] = jnp.dot(xb, w[0] - w[2],
                               preferred_element_type=jnp.float32)
        xw1_ref[...] = jnp.dot(xb, w[1], preferred_element_type=jnp.float32)
        u1_ref[...] = dinv * jnp.dot(xb, w[2],
                                     preferred_element_type=jnp.float32)
        dinv_ref[...] = dinv

    o = jax.ShapeDtypeStruct((N, HID), jnp.float32)
    return pl.pallas_call(
        body,
        grid=(GRID,),
        in_specs=[_row_spec(F_IN), _full_spec((3, F_IN, HID)), _parts_spec(1)],
        out_specs=[_row_spec(HID), _row_spec(HID), _row_spec(HID),
                   _row_spec(1)],
        out_shape=[o, o, o, jax.ShapeDtypeStruct((N, 1), jnp.float32)],
    )(x, w1, parts)


def _tc_mid(d):
    def body(a_ref, p_ref, dinv_ref, v_ref):
        g = p_ref[...]
        dinv = dinv_ref[...]
        v_ref[...] = dinv * (a_ref[...] - 2.0 * dinv * g)

    def run(a, parts, dinv):
        return pl.pallas_call(
            body,
            grid=(GRID,),
            in_specs=[_row_spec(d), _row_spec(d), _row_spec(1)],
            out_specs=_row_spec(d),
            out_shape=jax.ShapeDtypeStruct((N, d), jnp.float32),
        )(a, parts, dinv)
    return run


def _tc3(xw0, parts, dinv, b1, w2):
    def body(xw0_ref, p_ref, dinv_ref, b_ref, w_ref,
             y0_ref, y1_ref, u2_ref):
        g = p_ref[...]
        dinv = dinv_ref[...]
        h = jax.nn.relu(xw0_ref[...] - dinv * g + b_ref[...])
        w = w_ref[...]
        y0_ref[...] = jnp.dot(h, w[0] - w[2],
                              preferred_element_type=jnp.float32)
        y1_ref[...] = jnp.dot(h, w[1], preferred_element_type=jnp.float32)
        u2_ref[...] = dinv * jnp.dot(h, w[2],
                                     preferred_element_type=jnp.float32)

    o = jax.ShapeDtypeStruct((N, CLS), jnp.float32)
    return pl.pallas_call(
        body,
        grid=(GRID,),
        in_specs=[_row_spec(HID), _row_spec(HID), _row_spec(1),
                  _full_spec((1, HID)), _full_spec((3, HID, CLS))],
        out_specs=[_row_spec(CLS), _row_spec(CLS), _row_spec(CLS)],
        out_shape=[o, o, o],
    )(xw0, parts, dinv, b1, w2)


def _tc5(y0, parts, dinv, b2):
    def body(y0_ref, p_ref, dinv_ref, b_ref, out_ref):
        z = y0_ref[...] - dinv_ref[...] * p_ref[...] + b_ref[...]
        m = jnp.max(z, axis=1, keepdims=True)
        e = jnp.exp(z - m)
        out_ref[...] = (z - m) - jnp.log(jnp.sum(e, axis=1, keepdims=True))

    return pl.pallas_call(
        body,
        grid=(GRID,),
        in_specs=[_row_spec(CLS), _row_spec(CLS), _row_spec(1),
                  _full_spec((1, CLS))],
        out_specs=_row_spec(CLS),
        out_shape=jax.ShapeDtypeStruct((N, CLS), jnp.float32),
    )(y0, parts, dinv, b2)


def kernel(x, edge_index, W1, b1, W2, b2):
    e = edge_index.shape[1]
    nch2 = -(-e // (NS * CH))
    nch2 = -(-nch2 // NBUF) * NBUF
    e_pad = NS * CH * nch2
    row = jnp.pad(edge_index[0], (0, e_pad - e)).reshape(e_pad // CH, CH)
    col = jnp.pad(edge_index[1], (0, e_pad - e)).reshape(e_pad // CH, CH)

    degp = _make_deg(nch2 // NC)(row, col).reshape(NC, N_ACC, 1)
    xw0, xw1, u1, dinv = _tc1(x, W1, degp)

    prop64 = _make_prop(HID, nch2)
    prop32 = _make_prop(CLS, nch2)

    g1 = prop64(u1, row, col).reshape(N_ACC, HID)
    v1 = _tc_mid(HID)(xw1, g1, dinv)
    g2 = prop64(v1, row, col).reshape(N_ACC, HID)
    y0, y1, u2 = _tc3(xw0, g2, dinv, b1.reshape(1, HID), W2)
    g3 = prop32(u2, row, col).reshape(N_ACC, CLS)
    v2 = _tc_mid(CLS)(y1, g3, dinv)
    g4 = prop32(v2, row, col).reshape(N_ACC, CLS)
    return _tc5(y0, g4, dinv, b2.reshape(1, CLS))


# speedup vs baseline: 19.4008x; 1.2785x over previous
import functools

import jax
import jax.numpy as jnp
from jax import lax
from jax.experimental import pallas as pl
from jax.experimental.pallas import tpu as pltpu
from jax.experimental.pallas import tpu_sc as plsc

N = 10000
F_IN = 128
HID = 64
CLS = 32

NC = 2
NS = 16
L = 16
NW = NC * NS

CH = 128
N_ACC = 10240
R = N_ACC // NS
DUMMY = N

NBUF = 4

BLK = 400
GRID = N // BLK

_sc_mesh = functools.partial(
    plsc.VectorSubcoreMesh, core_axis_name="c", subcore_axis_name="s")


def _make_deg(nch):
    @functools.partial(
        pl.kernel,
        out_type=jax.ShapeDtypeStruct((NC, N_ACC), jnp.float32),
        mesh=_sc_mesh(),
        compiler_params=pltpu.CompilerParams(needs_layout_passes=False, use_tc_tiling_on_sc=False),
        scratch_types=[
            pltpu.VMEM((nch, CH), jnp.int32),
            pltpu.VMEM((nch, CH), jnp.int32),
            pltpu.VMEM((N_ACC,), jnp.float32),
            pltpu.VMEM((R,), jnp.float32),
            pltpu.VMEM((R,), jnp.float32),
            pltpu.VMEM_SHARED((NS, N_ACC), jnp.float32),
        ],
    )
    def deg(row_hbm, col_hbm, out_hbm, rows_v, cols_v, hist, accv, tmpv, sh):
        c = lax.axis_index("c")
        s = lax.axis_index("s")
        wid = c * NS + s
        zero16 = jnp.zeros((L,), jnp.float32)

        def z(i, _):
            hist[pl.ds(i * L, L)] = zero16
            return 0
        lax.fori_loop(0, N_ACC // L, z, 0)

        pltpu.sync_copy(row_hbm.at[pl.ds(wid * nch, nch)], rows_v)
        pltpu.sync_copy(col_hbm.at[pl.ds(wid * nch, nch)], cols_v)

        ones16 = jnp.ones((L,), jnp.float32)

        def count(j, _):
            for k in range(CH // L):
                r = rows_v[j, pl.ds(k * L, L)]
                cc = cols_v[j, pl.ds(k * L, L)]
                plsc.addupdate_scatter(hist, [r], ones16, mask=r != cc)
            return 0
        lax.fori_loop(0, nch, count, 0)

        pltpu.sync_copy(hist, sh.at[s])
        plsc.subcore_barrier()

        pltpu.sync_copy(sh.at[0, pl.ds(s * R, R)], accv)
        for t in range(1, NS):
            pltpu.sync_copy(sh.at[t, pl.ds(s * R, R)], tmpv)

            def addl(i, _):
                accv[pl.ds(i * L, L)] = (accv[pl.ds(i * L, L)]
                                         + tmpv[pl.ds(i * L, L)])
                return 0
            lax.fori_loop(0, R // L, addl, 0)
        pltpu.sync_copy(accv, out_hbm.at[c, pl.ds(s * R, R)])

    return deg


def _make_prop(d, nch2):
    hd = d // NC

    @functools.partial(
        pl.kernel,
        out_type=jax.ShapeDtypeStruct((N_ACC, d), jnp.float32),
        mesh=_sc_mesh(),
        compiler_params=pltpu.CompilerParams(needs_layout_passes=False, use_tc_tiling_on_sc=False),
        scratch_types=[
            pltpu.VMEM((nch2, CH), jnp.int32),
            pltpu.VMEM((nch2, CH), jnp.int32),
            [pltpu.VMEM((CH, hd), jnp.float32) for _ in range(NBUF)],
            [pltpu.SemaphoreType.DMA for _ in range(NBUF)],
            [pltpu.SemaphoreType.DMA for _ in range(NBUF)],
            pltpu.VMEM_SHARED((N_ACC, hd), jnp.float32),
            pltpu.VMEM_SHARED((N, hd), jnp.float32),
        ],
    )
    def prop(u_hbm, row_hbm, col_hbm, out_hbm,
             rows_v, cols_v, gb, sem_g, sem_s, acc, u_sh):
        c = lax.axis_index("c")
        s = lax.axis_index("s")
        zero16 = jnp.zeros((L,), jnp.float32)

        nrs = N // NS
        pltpu.sync_copy(u_hbm.at[pl.ds(s * nrs, nrs), pl.ds(c * hd, hd)],
                        u_sh.at[pl.ds(s * nrs, nrs)])

        def zrow(i, _):
            for k in range(hd // L):
                gb[0][i, pl.ds(k * L, L)] = zero16
            return 0
        lax.fori_loop(0, CH, zrow, 0)
        for t in range(R // CH):
            pltpu.sync_copy(gb[0], acc.at[pl.ds(s * R + t * CH, CH)])

        pltpu.sync_copy(row_hbm.at[pl.ds(s * nch2, nch2)], rows_v)
        pltpu.sync_copy(col_hbm.at[pl.ds(s * nch2, nch2)], cols_v)

        def fix(j, _):
            for k in range(CH // L):
                r = rows_v[j, pl.ds(k * L, L)]
                cc = cols_v[j, pl.ds(k * L, L)]
                cols_v[j, pl.ds(k * L, L)] = jnp.where(r == cc, DUMMY, cc)
            return 0
        lax.fori_loop(0, nch2, fix, 0)

        plsc.subcore_barrier()

        def group(p, _):
            j0 = p * NBUF
            for b in range(NBUF):
                jb = j0 + b

                @pl.when(p > 0)
                def _():
                    pltpu.make_async_copy(
                        gb[b], acc.at[cols_v.at[jb]], sem_s[b]).wait()
                pltpu.async_copy(u_sh.at[rows_v.at[jb]], gb[b], sem_g[b])
            for b in range(NBUF):
                jb = j0 + b
                pltpu.make_async_copy(
                    u_sh.at[rows_v.at[jb]], gb[b], sem_g[b]).wait()
                pltpu.async_copy(
                    gb[b], acc.at[cols_v.at[jb]], sem_s[b], add=True)
            return 0
        lax.fori_loop(0, nch2 // NBUF, group, 0)
        for b in range(NBUF):
            pltpu.make_async_copy(gb[b], acc.at[cols_v.at[b]], sem_s[b]).wait()

        plsc.subcore_barrier()
        pltpu.sync_copy(acc.at[pl.ds(s * R, R)],
                        out_hbm.at[pl.ds(s * R, R), pl.ds(c * hd, hd)])

    return prop


def _row_spec(d):
    return pl.BlockSpec((BLK, d), lambda i: (i, 0))


def _parts_spec(d):
    return pl.BlockSpec((NC, BLK, d), lambda i: (0, i, 0))


def _full_spec(shape):
    nd = len(shape)
    return pl.BlockSpec(shape, lambda i, _nd=nd: (0,) * _nd)


def _tc1(x, w1, parts):
    def body(x_ref, w_ref, p_ref, xw0_ref, xw1_ref, u1_ref, dinv_ref):
        degb = p_ref[0] + p_ref[1]
        dinv = jnp.where(degb > 0,
                         lax.rsqrt(jnp.where(degb > 0, degb, 1.0)), 0.0)
        xb = x_ref[...]
        w = w_ref[...]
        xw0_ref[...] = jnp.dot(xb, w[0] - w[2],
                               preferred_element_type=jnp.float32)
        xw1_ref[...] = jnp.dot(xb, w[1], preferred_element_type=jnp.float32)
        u1_ref[...] = dinv * jnp.dot(xb, w[2],
                                     preferred_element_type=jnp.float32)
        dinv_ref[...] = dinv

    o = jax.ShapeDtypeStruct((N, HID), jnp.float32)
    return pl.pallas_call(
        body,
        grid=(GRID,),
        in_specs=[_row_spec(F_IN), _full_spec((3, F_IN, HID)), _parts_spec(1)],
        out_specs=[_row_spec(HID), _row_spec(HID), _row_spec(HID),
                   _row_spec(1)],
        out_shape=[o, o, o, jax.ShapeDtypeStruct((N, 1), jnp.float32)],
    )(x, w1, parts)


def _tc_mid(d):
    def body(a_ref, p_ref, dinv_ref, v_ref):
        g = p_ref[...]
        dinv = dinv_ref[...]
        v_ref[...] = dinv * (a_ref[...] - 2.0 * dinv * g)

    def run(a, parts, dinv):
        return pl.pallas_call(
            body,
            grid=(GRID,),
            in_specs=[_row_spec(d), _row_spec(d), _row_spec(1)],
            out_specs=_row_spec(d),
            out_shape=jax.ShapeDtypeStruct((N, d), jnp.float32),
        )(a, parts, dinv)
    return run


def _tc3(xw0, parts, dinv, b1, w2):
    def body(xw0_ref, p_ref, dinv_ref, b_ref, w_ref,
             y0_ref, y1_ref, u2_ref):
        g = p_ref[...]
        dinv = dinv_ref[...]
        h = jax.nn.relu(xw0_ref[...] - dinv * g + b_ref[...])
        w = w_ref[...]
        y0_ref[...] = jnp.dot(h, w[0] - w[2],
                              preferred_element_type=jnp.float32)
        y1_ref[...] = jnp.dot(h, w[1], preferred_element_type=jnp.float32)
        u2_ref[...] = dinv * jnp.dot(h, w[2],
                                     preferred_element_type=jnp.float32)

    o = jax.ShapeDtypeStruct((N, CLS), jnp.float32)
    return pl.pallas_call(
        body,
        grid=(GRID,),
        in_specs=[_row_spec(HID), _row_spec(HID), _row_spec(1),
                  _full_spec((1, HID)), _full_spec((3, HID, CLS))],
        out_specs=[_row_spec(CLS), _row_spec(CLS), _row_spec(CLS)],
        out_shape=[o, o, o],
    )(xw0, parts, dinv, b1, w2)


def _tc5(y0, parts, dinv, b2):
    def body(y0_ref, p_ref, dinv_ref, b_ref, out_ref):
        z = y0_ref[...] - dinv_ref[...] * p_ref[...] + b_ref[...]
        m = jnp.max(z, axis=1, keepdims=True)
        e = jnp.exp(z - m)
        out_ref[...] = (z - m) - jnp.log(jnp.sum(e, axis=1, keepdims=True))

    return pl.pallas_call(
        body,
        grid=(GRID,),
        in_specs=[_row_spec(CLS), _row_spec(CLS), _row_spec(1),
                  _full_spec((1, CLS))],
        out_specs=_row_spec(CLS),
        out_shape=jax.ShapeDtypeStruct((N, CLS), jnp.float32),
    )(y0, parts, dinv, b2)


def kernel(x, edge_index, W1, b1, W2, b2):
    e = edge_index.shape[1]
    nch2 = -(-e // (NS * CH))
    nch2 = -(-nch2 // NBUF) * NBUF
    e_pad = NS * CH * nch2
    row = jnp.pad(edge_index[0], (0, e_pad - e)).reshape(e_pad // CH, CH)
    col = jnp.pad(edge_index[1], (0, e_pad - e)).reshape(e_pad // CH, CH)

    degp = _make_deg(nch2 // NC)(row, col).reshape(NC, N_ACC, 1)
    xw0, xw1, u1, dinv = _tc1(x, W1, degp)

    prop64 = _make_prop(HID, nch2)
    prop32 = _make_prop(CLS, nch2)

    g1 = prop64(u1, row, col)
    v1 = _tc_mid(HID)(xw1, g1, dinv)
    g2 = prop64(v1, row, col)
    y0, y1, u2 = _tc3(xw0, g2, dinv, b1.reshape(1, HID), W2)
    g3 = prop32(u2, row, col)
    v2 = _tc_mid(CLS)(y1, g3, dinv)
    g4 = prop32(v2, row, col)
    return _tc5(y0, g4, dinv, b2.reshape(1, CLS))


# speedup vs baseline: 21.2039x; 1.0929x over previous
import functools

import jax
import jax.numpy as jnp
from jax import lax
from jax.experimental import pallas as pl
from jax.experimental.pallas import tpu as pltpu
from jax.experimental.pallas import tpu_sc as plsc

N = 10000
F_IN = 128
HID = 64
CLS = 32

NC = 2
NS = 16
L = 16
NW = NC * NS

CH = 128
N_ACC = 10240
R = N_ACC // NS
DUMMY = N

NBUF = 4

BLK = 400
GRID = N // BLK

_sc_mesh = functools.partial(
    plsc.VectorSubcoreMesh, core_axis_name="c", subcore_axis_name="s")


def _make_deg(nch):
    @functools.partial(
        pl.kernel,
        out_type=(jax.ShapeDtypeStruct((NC, N_ACC), jnp.float32),
                  jax.ShapeDtypeStruct((NC, NS, N_ACC), jnp.float32)),
        mesh=_sc_mesh(),
        compiler_params=pltpu.CompilerParams(needs_layout_passes=False, use_tc_tiling_on_sc=False),
        scratch_types=[
            pltpu.VMEM((nch, CH), jnp.int32),
            pltpu.VMEM((nch, CH), jnp.int32),
            pltpu.VMEM((N_ACC,), jnp.float32),
            pltpu.VMEM((R,), jnp.float32),
            pltpu.VMEM((R,), jnp.float32),
        ],
    )
    def deg(row_hbm, col_hbm, out_hbm, sh, rows_v, cols_v, hist, accv, tmpv):
        c = lax.axis_index("c")
        s = lax.axis_index("s")
        wid = c * NS + s
        zero16 = jnp.zeros((L,), jnp.float32)

        def z(i, _):
            hist[pl.ds(i * L, L)] = zero16
            return 0
        lax.fori_loop(0, N_ACC // L, z, 0)

        pltpu.sync_copy(row_hbm.at[pl.ds(wid * nch, nch)], rows_v)
        pltpu.sync_copy(col_hbm.at[pl.ds(wid * nch, nch)], cols_v)

        ones16 = jnp.ones((L,), jnp.float32)

        def count(j, _):
            for k in range(CH // L):
                r = rows_v[j, pl.ds(k * L, L)]
                cc = cols_v[j, pl.ds(k * L, L)]
                plsc.addupdate_scatter(hist, [r], ones16, mask=r != cc)
            return 0
        lax.fori_loop(0, nch, count, 0)

        pltpu.sync_copy(hist, sh.at[c, s])
        plsc.subcore_barrier()

        pltpu.sync_copy(sh.at[c, 0, pl.ds(s * R, R)], accv)
        for t in range(1, NS):
            pltpu.sync_copy(sh.at[c, t, pl.ds(s * R, R)], tmpv)

            def addl(i, _):
                accv[pl.ds(i * L, L)] = (accv[pl.ds(i * L, L)]
                                         + tmpv[pl.ds(i * L, L)])
                return 0
            lax.fori_loop(0, R // L, addl, 0)
        pltpu.sync_copy(accv, out_hbm.at[c, pl.ds(s * R, R)])

    return deg


def _make_layer(d, nch2):
    hd = d // NC
    nrs = N // NS
    pch = 125

    @functools.partial(
        pl.kernel,
        out_type=jax.ShapeDtypeStruct((N_ACC, d), jnp.float32),
        mesh=_sc_mesh(),
        compiler_params=pltpu.CompilerParams(needs_layout_passes=False, use_tc_tiling_on_sc=False),
        scratch_types=[
            pltpu.VMEM((nch2, CH), jnp.int32),
            pltpu.VMEM((nch2, CH), jnp.int32),
            [pltpu.VMEM((CH, hd), jnp.float32) for _ in range(NBUF)],
            [pltpu.SemaphoreType.DMA for _ in range(NBUF)],
            [pltpu.SemaphoreType.DMA for _ in range(NBUF)],
            pltpu.VMEM((pch, hd), jnp.float32),
            pltpu.VMEM((pch, hd), jnp.float32),
            pltpu.VMEM((pch, L), jnp.float32),
            pltpu.VMEM_SHARED((N_ACC, hd), jnp.float32),
            pltpu.VMEM_SHARED((N, hd), jnp.float32),
        ],
    )
    def layer(u_hbm, a_hbm, dinv_hbm, row_hbm, col_hbm, out_hbm,
              rows_v, cols_v, gb, sem_g, sem_s, gvb, avb, dvb, acc, u_sh):
        c = lax.axis_index("c")
        s = lax.axis_index("s")
        zero16 = jnp.zeros((L,), jnp.float32)

        def zero_gb0(_=None):
            def zrow(i, __):
                for k in range(hd // L):
                    gb[0][i, pl.ds(k * L, L)] = zero16
                return 0
            lax.fori_loop(0, CH, zrow, 0)

        def zero_acc_slice(_=None):
            for t in range(R // CH):
                pltpu.sync_copy(gb[0], acc.at[pl.ds(s * R + t * CH, CH)])

        def prop_loop(_=None):
            def group(p, __):
                j0 = p * NBUF
                for b in range(NBUF):
                    jb = j0 + b

                    @pl.when(p > 0)
                    def _():
                        pltpu.make_async_copy(
                            gb[b], acc.at[cols_v.at[jb]], sem_s[b]).wait()
                    pltpu.async_copy(u_sh.at[rows_v.at[jb]], gb[b], sem_g[b])
                for b in range(NBUF):
                    jb = j0 + b
                    pltpu.make_async_copy(
                        u_sh.at[rows_v.at[jb]], gb[b], sem_g[b]).wait()
                    pltpu.async_copy(
                        gb[b], acc.at[cols_v.at[jb]], sem_s[b], add=True)
                return 0
            lax.fori_loop(0, nch2 // NBUF, group, 0)
            for b in range(NBUF):
                pltpu.make_async_copy(
                    gb[b], acc.at[cols_v.at[b]], sem_s[b]).wait()

        pltpu.sync_copy(u_hbm.at[pl.ds(s * nrs, nrs), pl.ds(c * hd, hd)],
                        u_sh.at[pl.ds(s * nrs, nrs)])
        zero_gb0()
        zero_acc_slice()
        pltpu.sync_copy(row_hbm.at[pl.ds(s * nch2, nch2)], rows_v)
        pltpu.sync_copy(col_hbm.at[pl.ds(s * nch2, nch2)], cols_v)

        def fix(j, _):
            for k in range(CH // L):
                r = rows_v[j, pl.ds(k * L, L)]
                cc = cols_v[j, pl.ds(k * L, L)]
                cols_v[j, pl.ds(k * L, L)] = jnp.where(r == cc, DUMMY, cc)
            return 0
        lax.fori_loop(0, nch2, fix, 0)
        plsc.subcore_barrier()
        prop_loop()
        plsc.subcore_barrier()

        def p2chunk(q, _):
            base = s * nrs + q * pch
            pltpu.sync_copy(acc.at[pl.ds(base, pch)], gvb)
            pltpu.sync_copy(a_hbm.at[pl.ds(base, pch), pl.ds(c * hd, hd)],
                            avb)
            pltpu.sync_copy(dinv_hbm.at[pl.ds(base, pch)], dvb)

            def scale(r, __):
                dv = dvb[r, pl.ds(0, L)]
                for k in range(hd // L):
                    g = gvb[r, pl.ds(k * L, L)]
                    a = avb[r, pl.ds(k * L, L)]
                    avb[r, pl.ds(k * L, L)] = dv * (a - 2.0 * dv * g)
                return 0
            lax.fori_loop(0, pch, scale, 0)
            pltpu.sync_copy(avb, u_sh.at[pl.ds(base, pch)])
            return 0
        lax.fori_loop(0, nrs // pch, p2chunk, 0)
        plsc.subcore_barrier()
        zero_gb0()
        zero_acc_slice()
        plsc.subcore_barrier()

        prop_loop()
        plsc.subcore_barrier()
        pltpu.sync_copy(acc.at[pl.ds(s * R, R)],
                        out_hbm.at[pl.ds(s * R, R), pl.ds(c * hd, hd)])

    return layer


def _row_spec(d):
    return pl.BlockSpec((BLK, d), lambda i: (i, 0))


def _parts_spec(d):
    return pl.BlockSpec((NC, BLK, d), lambda i: (0, i, 0))


def _full_spec(shape):
    nd = len(shape)
    return pl.BlockSpec(shape, lambda i, _nd=nd: (0,) * _nd)


def _tc1(x, w1, parts):
    def body(x_ref, w_ref, p_ref, xw0_ref, xw1_ref, u1_ref, dinv_ref):
        degb = p_ref[0] + p_ref[1]
        dinv = jnp.where(degb > 0,
                         lax.rsqrt(jnp.where(degb > 0, degb, 1.0)), 0.0)
        xb = x_ref[...]
        w = w_ref[...]
        xw0_ref[...] = jnp.dot(xb, w[0] - w[2],
                               preferred_element_type=jnp.float32)
        xw1_ref[...] = jnp.dot(xb, w[1], preferred_element_type=jnp.float32)
        u1_ref[...] = dinv * jnp.dot(xb, w[2],
                                     preferred_element_type=jnp.float32)
        dinv_ref[...] = jnp.broadcast_to(dinv, (BLK, 16))

    o = jax.ShapeDtypeStruct((N, HID), jnp.float32)
    return pl.pallas_call(
        body,
        grid=(GRID,),
        in_specs=[_row_spec(F_IN), _full_spec((3, F_IN, HID)), _parts_spec(1)],
        out_specs=[_row_spec(HID), _row_spec(HID), _row_spec(HID),
                   _row_spec(16)],
        out_shape=[o, o, o, jax.ShapeDtypeStruct((N, 16), jnp.float32)],
    )(x, w1, parts)


def _tc3(xw0, parts, dinv, b1, w2):
    def body(xw0_ref, p_ref, dinv_ref, b_ref, w_ref,
             y0_ref, y1_ref, u2_ref):
        g = p_ref[...]
        dinv = dinv_ref[:, :1]
        h = jax.nn.relu(xw0_ref[...] - dinv * g + b_ref[...])
        w = w_ref[...]
        y0_ref[...] = jnp.dot(h, w[0] - w[2],
                              preferred_element_type=jnp.float32)
        y1_ref[...] = jnp.dot(h, w[1], preferred_element_type=jnp.float32)
        u2_ref[...] = dinv * jnp.dot(h, w[2],
                                     preferred_element_type=jnp.float32)

    o = jax.ShapeDtypeStruct((N, CLS), jnp.float32)
    return pl.pallas_call(
        body,
        grid=(GRID,),
        in_specs=[_row_spec(HID), _row_spec(HID), _row_spec(16),
                  _full_spec((1, HID)), _full_spec((3, HID, CLS))],
        out_specs=[_row_spec(CLS), _row_spec(CLS), _row_spec(CLS)],
        out_shape=[o, o, o],
    )(xw0, parts, dinv, b1, w2)


def _tc5(y0, parts, dinv, b2):
    def body(y0_ref, p_ref, dinv_ref, b_ref, out_ref):
        z = y0_ref[...] - dinv_ref[:, :1] * p_ref[...] + b_ref[...]
        m = jnp.max(z, axis=1, keepdims=True)
        e = jnp.exp(z - m)
        out_ref[...] = (z - m) - jnp.log(jnp.sum(e, axis=1, keepdims=True))

    return pl.pallas_call(
        body,
        grid=(GRID,),
        in_specs=[_row_spec(CLS), _row_spec(CLS), _row_spec(16),
                  _full_spec((1, CLS))],
        out_specs=_row_spec(CLS),
        out_shape=jax.ShapeDtypeStruct((N, CLS), jnp.float32),
    )(y0, parts, dinv, b2)


def kernel(x, edge_index, W1, b1, W2, b2):
    e = edge_index.shape[1]
    nch2 = -(-e // (NS * CH))
    nch2 = -(-nch2 // NBUF) * NBUF
    e_pad = NS * CH * nch2
    row = jnp.pad(edge_index[0], (0, e_pad - e)).reshape(e_pad // CH, CH)
    col = jnp.pad(edge_index[1], (0, e_pad - e)).reshape(e_pad // CH, CH)

    degp = _make_deg(nch2 // NC)(row, col)[0].reshape(NC, N_ACC, 1)
    xw0, xw1, u1, dinv = _tc1(x, W1, degp)

    g2 = _make_layer(HID, nch2)(u1, xw1, dinv, row, col)
    y0, y1, u2 = _tc3(xw0, g2, dinv, b1.reshape(1, HID), W2)
    g4 = _make_layer(CLS, nch2)(u2, y1, dinv, row, col)
    return _tc5(y0, g4, dinv, b2.reshape(1, CLS))


# speedup vs baseline: 22.7126x; 1.0712x over previous
import functools

import jax
import jax.numpy as jnp
from jax import lax
from jax.experimental import pallas as pl
from jax.experimental.pallas import tpu as pltpu
from jax.experimental.pallas import tpu_sc as plsc

N = 10000
F_IN = 128
HID = 64
CLS = 32

NC = 2
NS = 16
L = 16
NW = NC * NS

CH = 128
N_ACC = 10240
R = N_ACC // NS
DUMMY = N

NBUF = 4

BLK = 2000
GRID = N // BLK

_sc_mesh = functools.partial(
    plsc.VectorSubcoreMesh, core_axis_name="c", subcore_axis_name="s")


def _make_deg(nch):
    @functools.partial(
        pl.kernel,
        out_type=(jax.ShapeDtypeStruct((NC, N_ACC), jnp.float32),
                  jax.ShapeDtypeStruct((NC, NS, N_ACC), jnp.float32)),
        mesh=_sc_mesh(),
        compiler_params=pltpu.CompilerParams(needs_layout_passes=False, use_tc_tiling_on_sc=False),
        scratch_types=[
            pltpu.VMEM((nch, CH), jnp.int32),
            pltpu.VMEM((nch, CH), jnp.int32),
            pltpu.VMEM((N_ACC,), jnp.float32),
            pltpu.VMEM((R,), jnp.float32),
            pltpu.VMEM((R,), jnp.float32),
        ],
    )
    def deg(row_hbm, col_hbm, out_hbm, sh, rows_v, cols_v, hist, accv, tmpv):
        c = lax.axis_index("c")
        s = lax.axis_index("s")
        wid = c * NS + s
        zero16 = jnp.zeros((L,), jnp.float32)

        def z(i, _):
            hist[pl.ds(i * L, L)] = zero16
            return 0
        lax.fori_loop(0, N_ACC // L, z, 0)

        pltpu.sync_copy(row_hbm.at[pl.ds(wid * nch, nch)], rows_v)
        pltpu.sync_copy(col_hbm.at[pl.ds(wid * nch, nch)], cols_v)

        ones16 = jnp.ones((L,), jnp.float32)

        def count(j, _):
            for k in range(CH // L):
                r = rows_v[j, pl.ds(k * L, L)]
                cc = cols_v[j, pl.ds(k * L, L)]
                plsc.addupdate_scatter(hist, [r], ones16, mask=r != cc)
            return 0
        lax.fori_loop(0, nch, count, 0)

        pltpu.sync_copy(hist, sh.at[c, s])
        plsc.subcore_barrier()

        pltpu.sync_copy(sh.at[c, 0, pl.ds(s * R, R)], accv)
        for t in range(1, NS):
            pltpu.sync_copy(sh.at[c, t, pl.ds(s * R, R)], tmpv)

            def addl(i, _):
                accv[pl.ds(i * L, L)] = (accv[pl.ds(i * L, L)]
                                         + tmpv[pl.ds(i * L, L)])
                return 0
            lax.fori_loop(0, R // L, addl, 0)
        pltpu.sync_copy(accv, out_hbm.at[c, pl.ds(s * R, R)])

    return deg


def _make_layer(d, nch2):
    hd = d // NC
    nrs = N // NS
    pch = 125

    @functools.partial(
        pl.kernel,
        out_type=jax.ShapeDtypeStruct((N_ACC, d), jnp.float32),
        mesh=_sc_mesh(),
        compiler_params=pltpu.CompilerParams(needs_layout_passes=False, use_tc_tiling_on_sc=False),
        scratch_types=[
            pltpu.VMEM((nch2, CH), jnp.int32),
            pltpu.VMEM((nch2, CH), jnp.int32),
            [pltpu.VMEM((CH, hd), jnp.float32) for _ in range(NBUF)],
            [pltpu.SemaphoreType.DMA for _ in range(NBUF)],
            [pltpu.SemaphoreType.DMA for _ in range(NBUF)],
            pltpu.VMEM((pch, hd), jnp.float32),
            pltpu.VMEM((pch, hd), jnp.float32),
            pltpu.VMEM((pch, L), jnp.float32),
            pltpu.VMEM_SHARED((N_ACC, hd), jnp.float32),
            pltpu.VMEM_SHARED((N, hd), jnp.float32),
        ],
    )
    def layer(u_hbm, a_hbm, dinv_hbm, row_hbm, col_hbm, out_hbm,
              rows_v, cols_v, gb, sem_g, sem_s, gvb, avb, dvb, acc, u_sh):
        c = lax.axis_index("c")
        s = lax.axis_index("s")
        zero16 = jnp.zeros((L,), jnp.float32)

        def zero_gb0(_=None):
            def zrow(i, __):
                for k in range(hd // L):
                    gb[0][i, pl.ds(k * L, L)] = zero16
                return 0
            lax.fori_loop(0, CH, zrow, 0)

        def zero_acc_slice(_=None):
            for t in range(R // CH):
                pltpu.sync_copy(gb[0], acc.at[pl.ds(s * R + t * CH, CH)])

        def prop_loop(_=None):
            def group(p, __):
                j0 = p * NBUF
                for b in range(NBUF):
                    jb = j0 + b

                    @pl.when(p > 0)
                    def _():
                        pltpu.make_async_copy(
                            gb[b], acc.at[cols_v.at[jb]], sem_s[b]).wait()
                    pltpu.async_copy(u_sh.at[rows_v.at[jb]], gb[b], sem_g[b])
                for b in range(NBUF):
                    jb = j0 + b
                    pltpu.make_async_copy(
                        u_sh.at[rows_v.at[jb]], gb[b], sem_g[b]).wait()
                    pltpu.async_copy(
                        gb[b], acc.at[cols_v.at[jb]], sem_s[b], add=True)
                return 0
            lax.fori_loop(0, nch2 // NBUF, group, 0)
            for b in range(NBUF):
                pltpu.make_async_copy(
                    gb[b], acc.at[cols_v.at[b]], sem_s[b]).wait()

        pltpu.sync_copy(u_hbm.at[pl.ds(s * nrs, nrs), pl.ds(c * hd, hd)],
                        u_sh.at[pl.ds(s * nrs, nrs)])
        zero_gb0()
        zero_acc_slice()
        pltpu.sync_copy(row_hbm.at[pl.ds(s * nch2, nch2)], rows_v)
        pltpu.sync_copy(col_hbm.at[pl.ds(s * nch2, nch2)], cols_v)

        def fix(j, _):
            for k in range(CH // L):
                r = rows_v[j, pl.ds(k * L, L)]
                cc = cols_v[j, pl.ds(k * L, L)]
                cols_v[j, pl.ds(k * L, L)] = jnp.where(r == cc, DUMMY, cc)
            return 0
        lax.fori_loop(0, nch2, fix, 0)
        plsc.subcore_barrier()
        prop_loop()
        plsc.subcore_barrier()

        def p2chunk(q, _):
            base = s * nrs + q * pch
            pltpu.sync_copy(acc.at[pl.ds(base, pch)], gvb)
            pltpu.sync_copy(a_hbm.at[pl.ds(base, pch), pl.ds(c * hd, hd)],
                            avb)
            pltpu.sync_copy(dinv_hbm.at[pl.ds(base, pch)], dvb)

            def scale(r, __):
                dv = dvb[r, pl.ds(0, L)]
                for k in range(hd // L):
                    g = gvb[r, pl.ds(k * L, L)]
                    a = avb[r, pl.ds(k * L, L)]
                    avb[r, pl.ds(k * L, L)] = dv * (a - 2.0 * dv * g)
                return 0
            lax.fori_loop(0, pch, scale, 0)
            pltpu.sync_copy(avb, u_sh.at[pl.ds(base, pch)])
            return 0
        lax.fori_loop(0, nrs // pch, p2chunk, 0)
        plsc.subcore_barrier()
        zero_gb0()
        zero_acc_slice()
        plsc.subcore_barrier()

        prop_loop()
        plsc.subcore_barrier()
        pltpu.sync_copy(acc.at[pl.ds(s * R, R)],
                        out_hbm.at[pl.ds(s * R, R), pl.ds(c * hd, hd)])

    return layer


def _row_spec(d):
    return pl.BlockSpec((BLK, d), lambda i: (i, 0))


def _parts_spec(d):
    return pl.BlockSpec((NC, BLK, d), lambda i: (0, i, 0))


def _full_spec(shape):
    nd = len(shape)
    return pl.BlockSpec(shape, lambda i, _nd=nd: (0,) * _nd)


def _tc1(x, w1, parts):
    def body(x_ref, w_ref, p_ref, xw0_ref, xw1_ref, u1_ref, dinv_ref):
        degb = p_ref[0] + p_ref[1]
        dinv = jnp.where(degb > 0,
                         lax.rsqrt(jnp.where(degb > 0, degb, 1.0)), 0.0)
        xb = x_ref[...]
        w = w_ref[...]
        xw0_ref[...] = jnp.dot(xb, w[0] - w[2],
                               preferred_element_type=jnp.float32)
        xw1_ref[...] = jnp.dot(xb, w[1], preferred_element_type=jnp.float32)
        u1_ref[...] = dinv * jnp.dot(xb, w[2],
                                     preferred_element_type=jnp.float32)
        dinv_ref[...] = jnp.broadcast_to(dinv, (BLK, 16))

    o = jax.ShapeDtypeStruct((N, HID), jnp.float32)
    return pl.pallas_call(
        body,
        grid=(GRID,),
        in_specs=[_row_spec(F_IN), _full_spec((3, F_IN, HID)), _parts_spec(1)],
        out_specs=[_row_spec(HID), _row_spec(HID), _row_spec(HID),
                   _row_spec(16)],
        out_shape=[o, o, o, jax.ShapeDtypeStruct((N, 16), jnp.float32)],
    )(x, w1, parts)


def _tc3(xw0, parts, dinv, b1, w2):
    def body(xw0_ref, p_ref, dinv_ref, b_ref, w_ref,
             y0_ref, y1_ref, u2_ref):
        g = p_ref[...]
        dinv = dinv_ref[:, :1]
        h = jax.nn.relu(xw0_ref[...] - dinv * g + b_ref[...])
        w = w_ref[...]
        y0_ref[...] = jnp.dot(h, w[0] - w[2],
                              preferred_element_type=jnp.float32)
        y1_ref[...] = jnp.dot(h, w[1], preferred_element_type=jnp.float32)
        u2_ref[...] = dinv * jnp.dot(h, w[2],
                                     preferred_element_type=jnp.float32)

    o = jax.ShapeDtypeStruct((N, CLS), jnp.float32)
    return pl.pallas_call(
        body,
        grid=(GRID,),
        in_specs=[_row_spec(HID), _row_spec(HID), _row_spec(16),
                  _full_spec((1, HID)), _full_spec((3, HID, CLS))],
        out_specs=[_row_spec(CLS), _row_spec(CLS), _row_spec(CLS)],
        out_shape=[o, o, o],
    )(xw0, parts, dinv, b1, w2)


def _tc5(y0, parts, dinv, b2):
    def body(y0_ref, p_ref, dinv_ref, b_ref, out_ref):
        z = y0_ref[...] - dinv_ref[:, :1] * p_ref[...] + b_ref[...]
        m = jnp.max(z, axis=1, keepdims=True)
        e = jnp.exp(z - m)
        out_ref[...] = (z - m) - jnp.log(jnp.sum(e, axis=1, keepdims=True))

    return pl.pallas_call(
        body,
        grid=(GRID,),
        in_specs=[_row_spec(CLS), _row_spec(CLS), _row_spec(16),
                  _full_spec((1, CLS))],
        out_specs=_row_spec(CLS),
        out_shape=jax.ShapeDtypeStruct((N, CLS), jnp.float32),
    )(y0, parts, dinv, b2)


def kernel(x, edge_index, W1, b1, W2, b2):
    e = edge_index.shape[1]
    nch2 = -(-e // (NS * CH))
    nch2 = -(-nch2 // NBUF) * NBUF
    e_pad = NS * CH * nch2
    row = jnp.pad(edge_index[0], (0, e_pad - e)).reshape(e_pad // CH, CH)
    col = jnp.pad(edge_index[1], (0, e_pad - e)).reshape(e_pad // CH, CH)

    degp = _make_deg(nch2 // NC)(row, col)[0].reshape(NC, N_ACC, 1)
    xw0, xw1, u1, dinv = _tc1(x, W1, degp)

    g2 = _make_layer(HID, nch2)(u1, xw1, dinv, row, col)
    y0, y1, u2 = _tc3(xw0, g2, dinv, b1.reshape(1, HID), W2)
    g4 = _make_layer(CLS, nch2)(u2, y1, dinv, row, col)
    return _tc5(y0, g4, dinv, b2.reshape(1, CLS))


# speedup vs baseline: 23.4659x; 1.0332x over previous
import functools

import jax
import jax.numpy as jnp
from jax import lax
from jax.experimental import pallas as pl
from jax.experimental.pallas import tpu as pltpu
from jax.experimental.pallas import tpu_sc as plsc

N = 10000
F_IN = 128
HID = 64
CLS = 32

NC = 2
NS = 16
L = 16
NW = NC * NS

CH = 128
N_ACC = 10240
R = N_ACC // NS
DUMMY = N

NBUF = 8

BLK = 2000
GRID = N // BLK

_sc_mesh = functools.partial(
    plsc.VectorSubcoreMesh, core_axis_name="c", subcore_axis_name="s")


def _make_deg(nch):
    @functools.partial(
        pl.kernel,
        out_type=(jax.ShapeDtypeStruct((NC, N_ACC), jnp.float32),
                  jax.ShapeDtypeStruct((NC, NS, N_ACC), jnp.float32)),
        mesh=_sc_mesh(),
        compiler_params=pltpu.CompilerParams(needs_layout_passes=False, use_tc_tiling_on_sc=False),
        scratch_types=[
            pltpu.VMEM((nch, CH), jnp.int32),
            pltpu.VMEM((nch, CH), jnp.int32),
            pltpu.VMEM((N_ACC,), jnp.float32),
            pltpu.VMEM((R,), jnp.float32),
            pltpu.VMEM((R,), jnp.float32),
        ],
    )
    def deg(row_hbm, col_hbm, out_hbm, sh, rows_v, cols_v, hist, accv, tmpv):
        c = lax.axis_index("c")
        s = lax.axis_index("s")
        wid = c * NS + s
        zero16 = jnp.zeros((L,), jnp.float32)

        def z(i, _):
            hist[pl.ds(i * L, L)] = zero16
            return 0
        lax.fori_loop(0, N_ACC // L, z, 0)

        pltpu.sync_copy(row_hbm.at[pl.ds(wid * nch, nch)], rows_v)
        pltpu.sync_copy(col_hbm.at[pl.ds(wid * nch, nch)], cols_v)

        ones16 = jnp.ones((L,), jnp.float32)

        def count(j, _):
            for k in range(CH // L):
                r = rows_v[j, pl.ds(k * L, L)]
                cc = cols_v[j, pl.ds(k * L, L)]
                plsc.addupdate_scatter(hist, [r], ones16, mask=r != cc)
            return 0
        lax.fori_loop(0, nch, count, 0)

        pltpu.sync_copy(hist, sh.at[c, s])
        plsc.subcore_barrier()

        pltpu.sync_copy(sh.at[c, 0, pl.ds(s * R, R)], accv)
        for t in range(1, NS):
            pltpu.sync_copy(sh.at[c, t, pl.ds(s * R, R)], tmpv)

            def addl(i, _):
                accv[pl.ds(i * L, L)] = (accv[pl.ds(i * L, L)]
                                         + tmpv[pl.ds(i * L, L)])
                return 0
            lax.fori_loop(0, R // L, addl, 0)
        pltpu.sync_copy(accv, out_hbm.at[c, pl.ds(s * R, R)])

    return deg


def _make_layer(d, nch2):
    hd = d // NC
    nrs = N // NS
    pch = 125

    @functools.partial(
        pl.kernel,
        out_type=jax.ShapeDtypeStruct((N_ACC, d), jnp.float32),
        mesh=_sc_mesh(),
        compiler_params=pltpu.CompilerParams(needs_layout_passes=False, use_tc_tiling_on_sc=False),
        scratch_types=[
            pltpu.VMEM((nch2, CH), jnp.int32),
            pltpu.VMEM((nch2, CH), jnp.int32),
            [pltpu.VMEM((CH, hd), jnp.float32) for _ in range(NBUF)],
            [pltpu.SemaphoreType.DMA for _ in range(NBUF)],
            [pltpu.SemaphoreType.DMA for _ in range(NBUF)],
            pltpu.VMEM((pch, hd), jnp.float32),
            pltpu.VMEM((pch, hd), jnp.float32),
            pltpu.VMEM((pch, L), jnp.float32),
            pltpu.VMEM_SHARED((N_ACC, hd), jnp.float32),
            pltpu.VMEM_SHARED((N, hd), jnp.float32),
        ],
    )
    def layer(u_hbm, a_hbm, dinv_hbm, row_hbm, col_hbm, out_hbm,
              rows_v, cols_v, gb, sem_g, sem_s, gvb, avb, dvb, acc, u_sh):
        c = lax.axis_index("c")
        s = lax.axis_index("s")
        zero16 = jnp.zeros((L,), jnp.float32)

        def zero_gb0(_=None):
            def zrow(i, __):
                for k in range(hd // L):
                    gb[0][i, pl.ds(k * L, L)] = zero16
                return 0
            lax.fori_loop(0, CH, zrow, 0)

        def zero_acc_slice(_=None):
            for t in range(R // CH):
                pltpu.sync_copy(gb[0], acc.at[pl.ds(s * R + t * CH, CH)])

        def prop_loop(_=None):
            def group(p, __):
                j0 = p * NBUF
                for b in range(NBUF):
                    jb = j0 + b

                    @pl.when(p > 0)
                    def _():
                        pltpu.make_async_copy(
                            gb[b], acc.at[cols_v.at[jb]], sem_s[b]).wait()
                    pltpu.async_copy(u_sh.at[rows_v.at[jb]], gb[b], sem_g[b])
                for b in range(NBUF):
                    jb = j0 + b
                    pltpu.make_async_copy(
                        u_sh.at[rows_v.at[jb]], gb[b], sem_g[b]).wait()
                    pltpu.async_copy(
                        gb[b], acc.at[cols_v.at[jb]], sem_s[b], add=True)
                return 0
            lax.fori_loop(0, nch2 // NBUF, group, 0)
            for b in range(NBUF):
                pltpu.make_async_copy(
                    gb[b], acc.at[cols_v.at[b]], sem_s[b]).wait()

        pltpu.sync_copy(u_hbm.at[pl.ds(s * nrs, nrs), pl.ds(c * hd, hd)],
                        u_sh.at[pl.ds(s * nrs, nrs)])
        zero_gb0()
        zero_acc_slice()
        pltpu.sync_copy(row_hbm.at[pl.ds(s * nch2, nch2)], rows_v)
        pltpu.sync_copy(col_hbm.at[pl.ds(s * nch2, nch2)], cols_v)

        def fix(j, _):
            for k in range(CH // L):
                r = rows_v[j, pl.ds(k * L, L)]
                cc = cols_v[j, pl.ds(k * L, L)]
                cols_v[j, pl.ds(k * L, L)] = jnp.where(r == cc, DUMMY, cc)
            return 0
        lax.fori_loop(0, nch2, fix, 0)
        plsc.subcore_barrier()
        prop_loop()
        plsc.subcore_barrier()

        def p2chunk(q, _):
            base = s * nrs + q * pch
            pltpu.sync_copy(acc.at[pl.ds(base, pch)], gvb)
            pltpu.sync_copy(a_hbm.at[pl.ds(base, pch), pl.ds(c * hd, hd)],
                            avb)
            pltpu.sync_copy(dinv_hbm.at[pl.ds(base, pch)], dvb)

            def scale(r, __):
                dv = dvb[r, pl.ds(0, L)]
                for k in range(hd // L):
                    g = gvb[r, pl.ds(k * L, L)]
                    a = avb[r, pl.ds(k * L, L)]
                    avb[r, pl.ds(k * L, L)] = dv * (a - 2.0 * dv * g)
                return 0
            lax.fori_loop(0, pch, scale, 0)
            pltpu.sync_copy(avb, u_sh.at[pl.ds(base, pch)])
            return 0
        lax.fori_loop(0, nrs // pch, p2chunk, 0)
        plsc.subcore_barrier()
        zero_gb0()
        zero_acc_slice()
        plsc.subcore_barrier()

        prop_loop()
        plsc.subcore_barrier()
        pltpu.sync_copy(acc.at[pl.ds(s * R, R)],
                        out_hbm.at[pl.ds(s * R, R), pl.ds(c * hd, hd)])

    return layer


def _row_spec(d):
    return pl.BlockSpec((BLK, d), lambda i: (i, 0))


def _parts_spec(d):
    return pl.BlockSpec((NC, BLK, d), lambda i: (0, i, 0))


def _full_spec(shape):
    nd = len(shape)
    return pl.BlockSpec(shape, lambda i, _nd=nd: (0,) * _nd)


def _tc1(x, w1, parts):
    def body(x_ref, w_ref, p_ref, xw0_ref, xw1_ref, u1_ref, dinv_ref):
        degb = p_ref[0] + p_ref[1]
        dinv = jnp.where(degb > 0,
                         lax.rsqrt(jnp.where(degb > 0, degb, 1.0)), 0.0)
        xb = x_ref[...]
        w = w_ref[...]
        xw0_ref[...] = jnp.dot(xb, w[0] - w[2],
                               preferred_element_type=jnp.float32)
        xw1_ref[...] = jnp.dot(xb, w[1], preferred_element_type=jnp.float32)
        u1_ref[...] = dinv * jnp.dot(xb, w[2],
                                     preferred_element_type=jnp.float32)
        dinv_ref[...] = jnp.broadcast_to(dinv, (BLK, 16))

    o = jax.ShapeDtypeStruct((N, HID), jnp.float32)
    return pl.pallas_call(
        body,
        grid=(GRID,),
        in_specs=[_row_spec(F_IN), _full_spec((3, F_IN, HID)), _parts_spec(1)],
        out_specs=[_row_spec(HID), _row_spec(HID), _row_spec(HID),
                   _row_spec(16)],
        out_shape=[o, o, o, jax.ShapeDtypeStruct((N, 16), jnp.float32)],
    )(x, w1, parts)


def _tc3(xw0, parts, dinv, b1, w2):
    def body(xw0_ref, p_ref, dinv_ref, b_ref, w_ref,
             y0_ref, y1_ref, u2_ref):
        g = p_ref[...]
        dinv = dinv_ref[:, :1]
        h = jax.nn.relu(xw0_ref[...] - dinv * g + b_ref[...])
        w = w_ref[...]
        y0_ref[...] = jnp.dot(h, w[0] - w[2],
                              preferred_element_type=jnp.float32)
        y1_ref[...] = jnp.dot(h, w[1], preferred_element_type=jnp.float32)
        u2_ref[...] = dinv * jnp.dot(h, w[2],
                                     preferred_element_type=jnp.float32)

    o = jax.ShapeDtypeStruct((N, CLS), jnp.float32)
    return pl.pallas_call(
        body,
        grid=(GRID,),
        in_specs=[_row_spec(HID), _row_spec(HID), _row_spec(16),
                  _full_spec((1, HID)), _full_spec((3, HID, CLS))],
        out_specs=[_row_spec(CLS), _row_spec(CLS), _row_spec(CLS)],
        out_shape=[o, o, o],
    )(xw0, parts, dinv, b1, w2)


def _tc5(y0, parts, dinv, b2):
    def body(y0_ref, p_ref, dinv_ref, b_ref, out_ref):
        z = y0_ref[...] - dinv_ref[:, :1] * p_ref[...] + b_ref[...]
        m = jnp.max(z, axis=1, keepdims=True)
        e = jnp.exp(z - m)
        out_ref[...] = (z - m) - jnp.log(jnp.sum(e, axis=1, keepdims=True))

    return pl.pallas_call(
        body,
        grid=(GRID,),
        in_specs=[_row_spec(CLS), _row_spec(CLS), _row_spec(16),
                  _full_spec((1, CLS))],
        out_specs=_row_spec(CLS),
        out_shape=jax.ShapeDtypeStruct((N, CLS), jnp.float32),
    )(y0, parts, dinv, b2)


def kernel(x, edge_index, W1, b1, W2, b2):
    e = edge_index.shape[1]
    nch2 = -(-e // (NS * CH))
    nch2 = -(-nch2 // NBUF) * NBUF
    e_pad = NS * CH * nch2
    row = jnp.pad(edge_index[0], (0, e_pad - e)).reshape(e_pad // CH, CH)
    col = jnp.pad(edge_index[1], (0, e_pad - e)).reshape(e_pad // CH, CH)

    degp = _make_deg(nch2 // NC)(row, col)[0].reshape(NC, N_ACC, 1)
    xw0, xw1, u1, dinv = _tc1(x, W1, degp)

    g2 = _make_layer(HID, nch2)(u1, xw1, dinv, row, col)
    y0, y1, u2 = _tc3(xw0, g2, dinv, b1.reshape(1, HID), W2)
    g4 = _make_layer(CLS, nch2)(u2, y1, dinv, row, col)
    return _tc5(y0, g4, dinv, b2.reshape(1, CLS))


# speedup vs baseline: 23.5581x; 1.0039x over previous
import functools

import jax
import jax.numpy as jnp
from jax import lax
from jax.experimental import pallas as pl
from jax.experimental.pallas import tpu as pltpu
from jax.experimental.pallas import tpu_sc as plsc

N = 10000
F_IN = 128
HID = 64
CLS = 32

NC = 2
NS = 16
L = 16
NW = NC * NS

CH = 128
N_ACC = 10240
R = N_ACC // NS
DUMMY = N

NBUF = 8

BLK = 2000
GRID = N // BLK

_sc_mesh = functools.partial(
    plsc.VectorSubcoreMesh, core_axis_name="c", subcore_axis_name="s")


def _make_deg(nch):
    @functools.partial(
        pl.kernel,
        out_type=(jax.ShapeDtypeStruct((NC, N_ACC), jnp.float32),
                  jax.ShapeDtypeStruct((NC, NS, N_ACC), jnp.float32)),
        mesh=_sc_mesh(),
        compiler_params=pltpu.CompilerParams(needs_layout_passes=False, use_tc_tiling_on_sc=False),
        scratch_types=[
            pltpu.VMEM((nch, CH), jnp.int32),
            pltpu.VMEM((nch, CH), jnp.int32),
            pltpu.VMEM((N_ACC,), jnp.float32),
            pltpu.VMEM((R,), jnp.float32),
            pltpu.VMEM((R,), jnp.float32),
        ],
    )
    def deg(row_hbm, col_hbm, out_hbm, sh, rows_v, cols_v, hist, accv, tmpv):
        c = lax.axis_index("c")
        s = lax.axis_index("s")
        wid = c * NS + s
        zero16 = jnp.zeros((L,), jnp.float32)

        def z(i, _):
            hist[pl.ds(i * L, L)] = zero16
            return 0
        lax.fori_loop(0, N_ACC // L, z, 0)

        pltpu.sync_copy(row_hbm.at[pl.ds(wid * nch, nch)], rows_v)
        pltpu.sync_copy(col_hbm.at[pl.ds(wid * nch, nch)], cols_v)

        ones16 = jnp.ones((L,), jnp.float32)

        def count(j, _):
            for k in range(CH // L):
                r = rows_v[j, pl.ds(k * L, L)]
                cc = cols_v[j, pl.ds(k * L, L)]
                plsc.addupdate_scatter(hist, [r], ones16, mask=r != cc)
            return 0
        lax.fori_loop(0, nch, count, 0)

        pltpu.sync_copy(hist, sh.at[c, s])
        plsc.subcore_barrier()

        pltpu.sync_copy(sh.at[c, 0, pl.ds(s * R, R)], accv)
        for t in range(1, NS):
            pltpu.sync_copy(sh.at[c, t, pl.ds(s * R, R)], tmpv)

            def addl(i, _):
                accv[pl.ds(i * L, L)] = (accv[pl.ds(i * L, L)]
                                         + tmpv[pl.ds(i * L, L)])
                return 0
            lax.fori_loop(0, R // L, addl, 0)
        pltpu.sync_copy(accv, out_hbm.at[c, pl.ds(s * R, R)])

    return deg


def _make_layer(d, nch2):
    hd = d // NC
    nrs = N // NS
    pch = 125

    @functools.partial(
        pl.kernel,
        out_type=jax.ShapeDtypeStruct((N_ACC, d), jnp.float32),
        mesh=_sc_mesh(),
        compiler_params=pltpu.CompilerParams(needs_layout_passes=False, use_tc_tiling_on_sc=False),
        scratch_types=[
            pltpu.VMEM((nch2, CH), jnp.int32),
            pltpu.VMEM((nch2, CH), jnp.int32),
            [pltpu.VMEM((CH, hd), jnp.float32) for _ in range(NBUF)],
            [pltpu.SemaphoreType.DMA for _ in range(NBUF)],
            [pltpu.SemaphoreType.DMA for _ in range(NBUF)],
            pltpu.VMEM((pch, hd), jnp.float32),
            pltpu.VMEM((pch, hd), jnp.float32),
            pltpu.VMEM((pch, L), jnp.float32),
            pltpu.VMEM_SHARED((N_ACC, hd), jnp.float32),
            pltpu.VMEM_SHARED((N, hd), jnp.float32),
        ],
    )
    def layer(u_hbm, a_hbm, dinv_hbm, row_hbm, col_hbm, out_hbm,
              rows_v, cols_v, gb, sem_g, sem_s, gvb, avb, dvb, acc, u_sh):
        c = lax.axis_index("c")
        s = lax.axis_index("s")
        zero16 = jnp.zeros((L,), jnp.float32)

        def zero_gb0(_=None):
            def zrow(i, __):
                for k in range(hd // L):
                    gb[0][i, pl.ds(k * L, L)] = zero16
                return 0
            lax.fori_loop(0, CH, zrow, 0)

        def zero_acc_slice(_=None):
            for t in range(R // CH):
                pltpu.sync_copy(gb[0], acc.at[pl.ds(s * R + t * CH, CH)])

        def prop_loop(_=None):
            def group(p, __):
                j0 = p * NBUF
                for b in range(NBUF):
                    jb = j0 + b

                    @pl.when(p > 0)
                    def _():
                        pltpu.make_async_copy(
                            gb[b], acc.at[cols_v.at[jb]], sem_s[b]).wait()
                    pltpu.async_copy(u_sh.at[rows_v.at[jb]], gb[b], sem_g[b])
                for b in range(NBUF):
                    jb = j0 + b
                    pltpu.make_async_copy(
                        u_sh.at[rows_v.at[jb]], gb[b], sem_g[b]).wait()
                    pltpu.async_copy(
                        gb[b], acc.at[cols_v.at[jb]], sem_s[b], add=True)
                return 0
            lax.fori_loop(0, nch2 // NBUF, group, 0)
            for b in range(NBUF):
                pltpu.make_async_copy(
                    gb[b], acc.at[cols_v.at[b]], sem_s[b]).wait()

        pltpu.sync_copy(u_hbm.at[pl.ds(s * nrs, nrs), pl.ds(c * hd, hd)],
                        u_sh.at[pl.ds(s * nrs, nrs)])
        zero_gb0()
        zero_acc_slice()
        pltpu.sync_copy(row_hbm.at[pl.ds(s * nch2, nch2)], rows_v)
        pltpu.sync_copy(col_hbm.at[pl.ds(s * nch2, nch2)], cols_v)

        def fix(j, _):
            for k in range(CH // L):
                r = rows_v[j, pl.ds(k * L, L)]
                cc = cols_v[j, pl.ds(k * L, L)]
                cols_v[j, pl.ds(k * L, L)] = jnp.where(r == cc, DUMMY, cc)
            return 0
        lax.fori_loop(0, nch2, fix, 0)
        plsc.subcore_barrier()
        prop_loop()
        plsc.subcore_barrier()

        def p2chunk(q, _):
            base = s * nrs + q * pch
            pltpu.sync_copy(acc.at[pl.ds(base, pch)], gvb)
            pltpu.sync_copy(a_hbm.at[pl.ds(base, pch), pl.ds(c * hd, hd)],
                            avb)
            pltpu.sync_copy(dinv_hbm.at[pl.ds(base, pch)], dvb)

            def scale(r, __):
                dv = dvb[r, pl.ds(0, L)]
                for k in range(hd // L):
                    g = gvb[r, pl.ds(k * L, L)]
                    a = avb[r, pl.ds(k * L, L)]
                    avb[r, pl.ds(k * L, L)] = dv * (a - 2.0 * dv * g)
                return 0
            lax.fori_loop(0, pch, scale, 0)
            pltpu.sync_copy(avb, u_sh.at[pl.ds(base, pch)])
            return 0
        lax.fori_loop(0, nrs // pch, p2chunk, 0)
        plsc.subcore_barrier()
        zero_gb0()
        zero_acc_slice()
        plsc.subcore_barrier()

        prop_loop()
        plsc.subcore_barrier()
        pltpu.sync_copy(acc.at[pl.ds(s * R, R)],
                        out_hbm.at[pl.ds(s * R, R), pl.ds(c * hd, hd)])

    return layer


def _row_spec(d):
    return pl.BlockSpec((BLK, d), lambda i: (i, 0))


def _parts_spec(d):
    return pl.BlockSpec((NC, BLK, d), lambda i: (0, i, 0))


def _full_spec(shape):
    nd = len(shape)
    return pl.BlockSpec(shape, lambda i, _nd=nd: (0,) * _nd)


def _tc1a(x, w1):
    def body(x_ref, w_ref, xw0_ref, xw1_ref, xw2_ref):
        xb = x_ref[...]
        w = w_ref[...]
        xw0_ref[...] = jnp.dot(xb, w[0] - w[2],
                               preferred_element_type=jnp.float32)
        xw1_ref[...] = jnp.dot(xb, w[1], preferred_element_type=jnp.float32)
        xw2_ref[...] = jnp.dot(xb, w[2], preferred_element_type=jnp.float32)

    o = jax.ShapeDtypeStruct((N, HID), jnp.float32)
    return pl.pallas_call(
        body,
        grid=(GRID,),
        in_specs=[_row_spec(F_IN), _full_spec((3, F_IN, HID))],
        out_specs=[_row_spec(HID), _row_spec(HID), _row_spec(HID)],
        out_shape=[o, o, o],
    )(x, w1)


def _tc1b(xw2, parts):
    def body(xw2_ref, p_ref, u1_ref, dinv_ref):
        degb = p_ref[0] + p_ref[1]
        dinv = jnp.where(degb > 0,
                         lax.rsqrt(jnp.where(degb > 0, degb, 1.0)), 0.0)
        u1_ref[...] = dinv * xw2_ref[...]
        dinv_ref[...] = jnp.broadcast_to(dinv, (BLK, 16))

    return pl.pallas_call(
        body,
        grid=(GRID,),
        in_specs=[_row_spec(HID), _parts_spec(1)],
        out_specs=[_row_spec(HID), _row_spec(16)],
        out_shape=[jax.ShapeDtypeStruct((N, HID), jnp.float32),
                   jax.ShapeDtypeStruct((N, 16), jnp.float32)],
    )(xw2, parts)


def _tc3(xw0, parts, dinv, b1, w2):
    def body(xw0_ref, p_ref, dinv_ref, b_ref, w_ref,
             y0_ref, y1_ref, u2_ref):
        g = p_ref[...]
        dinv = dinv_ref[:, :1]
        h = jax.nn.relu(xw0_ref[...] - dinv * g + b_ref[...])
        w = w_ref[...]
        y0_ref[...] = jnp.dot(h, w[0] - w[2],
                              preferred_element_type=jnp.float32)
        y1_ref[...] = jnp.dot(h, w[1], preferred_element_type=jnp.float32)
        u2_ref[...] = dinv * jnp.dot(h, w[2],
                                     preferred_element_type=jnp.float32)

    o = jax.ShapeDtypeStruct((N, CLS), jnp.float32)
    return pl.pallas_call(
        body,
        grid=(GRID,),
        in_specs=[_row_spec(HID), _row_spec(HID), _row_spec(16),
                  _full_spec((1, HID)), _full_spec((3, HID, CLS))],
        out_specs=[_row_spec(CLS), _row_spec(CLS), _row_spec(CLS)],
        out_shape=[o, o, o],
    )(xw0, parts, dinv, b1, w2)


def _tc5(y0, parts, dinv, b2):
    def body(y0_ref, p_ref, dinv_ref, b_ref, out_ref):
        z = y0_ref[...] - dinv_ref[:, :1] * p_ref[...] + b_ref[...]
        m = jnp.max(z, axis=1, keepdims=True)
        e = jnp.exp(z - m)
        out_ref[...] = (z - m) - jnp.log(jnp.sum(e, axis=1, keepdims=True))

    return pl.pallas_call(
        body,
        grid=(GRID,),
        in_specs=[_row_spec(CLS), _row_spec(CLS), _row_spec(16),
                  _full_spec((1, CLS))],
        out_specs=_row_spec(CLS),
        out_shape=jax.ShapeDtypeStruct((N, CLS), jnp.float32),
    )(y0, parts, dinv, b2)


def kernel(x, edge_index, W1, b1, W2, b2):
    e = edge_index.shape[1]
    nch2 = -(-e // (NS * CH))
    nch2 = -(-nch2 // NBUF) * NBUF
    e_pad = NS * CH * nch2
    row = jnp.pad(edge_index[0], (0, e_pad - e)).reshape(e_pad // CH, CH)
    col = jnp.pad(edge_index[1], (0, e_pad - e)).reshape(e_pad // CH, CH)

    xw0, xw1, xw2 = _tc1a(x, W1)
    degp = _make_deg(nch2 // NC)(row, col)[0].reshape(NC, N_ACC, 1)
    u1, dinv = _tc1b(xw2, degp)

    g2 = _make_layer(HID, nch2)(u1, xw1, dinv, row, col)
    y0, y1, u2 = _tc3(xw0, g2, dinv, b1.reshape(1, HID), W2)
    g4 = _make_layer(CLS, nch2)(u2, y1, dinv, row, col)
    return _tc5(y0, g4, dinv, b2.reshape(1, CLS))


# speedup vs baseline: 23.8139x; 1.0109x over previous
import functools

import jax
import jax.numpy as jnp
from jax import lax
from jax.experimental import pallas as pl
from jax.experimental.pallas import tpu as pltpu
from jax.experimental.pallas import tpu_sc as plsc

N = 10000
F_IN = 128
HID = 64
CLS = 32

NC = 2
NS = 16
L = 16
NW = NC * NS

CH = 128
N_ACC = 10240
R = N_ACC // NS
DUMMY = N

NBUF = 8

BLK = 2000
GRID = N // BLK

_sc_mesh = functools.partial(
    plsc.VectorSubcoreMesh, core_axis_name="c", subcore_axis_name="s")


def _make_deg(nch):
    @functools.partial(
        pl.kernel,
        out_type=(jax.ShapeDtypeStruct((NC, N_ACC), jnp.float32),
                  jax.ShapeDtypeStruct((NC, NS, N_ACC), jnp.float32),
                  jax.ShapeDtypeStruct((nch * NW, CH), jnp.int32)),
        mesh=_sc_mesh(),
        compiler_params=pltpu.CompilerParams(needs_layout_passes=False, use_tc_tiling_on_sc=False),
        scratch_types=[
            pltpu.VMEM((nch, CH), jnp.int32),
            pltpu.VMEM((nch, CH), jnp.int32),
            pltpu.VMEM((N_ACC,), jnp.float32),
            pltpu.VMEM((R,), jnp.float32),
            pltpu.VMEM((R,), jnp.float32),
        ],
    )
    def deg(row_hbm, col_hbm, out_hbm, sh, colf_hbm,
            rows_v, cols_v, hist, accv, tmpv):
        c = lax.axis_index("c")
        s = lax.axis_index("s")
        wid = c * NS + s
        zero16 = jnp.zeros((L,), jnp.float32)

        def z(i, _):
            hist[pl.ds(i * L, L)] = zero16
            return 0
        lax.fori_loop(0, N_ACC // L, z, 0)

        pltpu.sync_copy(row_hbm.at[pl.ds(wid * nch, nch)], rows_v)
        pltpu.sync_copy(col_hbm.at[pl.ds(wid * nch, nch)], cols_v)

        ones16 = jnp.ones((L,), jnp.float32)

        def count(j, _):
            for k in range(CH // L):
                r = rows_v[j, pl.ds(k * L, L)]
                cc = cols_v[j, pl.ds(k * L, L)]
                plsc.addupdate_scatter(hist, [r], ones16, mask=r != cc)
                cols_v[j, pl.ds(k * L, L)] = jnp.where(r == cc, DUMMY, cc)
            return 0
        lax.fori_loop(0, nch, count, 0)
        pltpu.sync_copy(cols_v, colf_hbm.at[pl.ds(wid * nch, nch)])

        pltpu.sync_copy(hist, sh.at[c, s])
        plsc.subcore_barrier()

        pltpu.sync_copy(sh.at[c, 0, pl.ds(s * R, R)], accv)
        for t in range(1, NS):
            pltpu.sync_copy(sh.at[c, t, pl.ds(s * R, R)], tmpv)

            def addl(i, _):
                accv[pl.ds(i * L, L)] = (accv[pl.ds(i * L, L)]
                                         + tmpv[pl.ds(i * L, L)])
                return 0
            lax.fori_loop(0, R // L, addl, 0)
        pltpu.sync_copy(accv, out_hbm.at[c, pl.ds(s * R, R)])

    return deg


def _make_layer(d, nch2):
    hd = d // NC
    nrs = N // NS
    pch = 125

    @functools.partial(
        pl.kernel,
        out_type=jax.ShapeDtypeStruct((N_ACC, d), jnp.float32),
        mesh=_sc_mesh(),
        compiler_params=pltpu.CompilerParams(needs_layout_passes=False, use_tc_tiling_on_sc=False),
        scratch_types=[
            pltpu.VMEM((nch2, CH), jnp.int32),
            pltpu.VMEM((nch2, CH), jnp.int32),
            [pltpu.VMEM((CH, hd), jnp.float32) for _ in range(NBUF)],
            [pltpu.SemaphoreType.DMA for _ in range(NBUF)],
            [pltpu.SemaphoreType.DMA for _ in range(NBUF)],
            pltpu.VMEM((pch, hd), jnp.float32),
            pltpu.VMEM((pch, hd), jnp.float32),
            pltpu.VMEM((pch, L), jnp.float32),
            pltpu.VMEM_SHARED((N_ACC, hd), jnp.float32),
            pltpu.VMEM_SHARED((N, hd), jnp.float32),
        ],
    )
    def layer(u_hbm, a_hbm, dinv_hbm, row_hbm, col_hbm, out_hbm,
              rows_v, cols_v, gb, sem_g, sem_s, gvb, avb, dvb, acc, u_sh):
        c = lax.axis_index("c")
        s = lax.axis_index("s")
        zero16 = jnp.zeros((L,), jnp.float32)

        def zero_gb0(_=None):
            def zrow(i, __):
                for k in range(hd // L):
                    gb[0][i, pl.ds(k * L, L)] = zero16
                return 0
            lax.fori_loop(0, CH, zrow, 0)

        def zero_acc_slice(_=None):
            for t in range(R // CH):
                pltpu.sync_copy(gb[0], acc.at[pl.ds(s * R + t * CH, CH)])

        def prop_loop(_=None):
            def group(p, __):
                j0 = p * NBUF
                for b in range(NBUF):
                    jb = j0 + b

                    @pl.when(p > 0)
                    def _():
                        pltpu.make_async_copy(
                            gb[b], acc.at[cols_v.at[jb]], sem_s[b]).wait()
                    pltpu.async_copy(u_sh.at[rows_v.at[jb]], gb[b], sem_g[b])
                for b in range(NBUF):
                    jb = j0 + b
                    pltpu.make_async_copy(
                        u_sh.at[rows_v.at[jb]], gb[b], sem_g[b]).wait()
                    pltpu.async_copy(
                        gb[b], acc.at[cols_v.at[jb]], sem_s[b], add=True)
                return 0
            lax.fori_loop(0, nch2 // NBUF, group, 0)
            for b in range(NBUF):
                pltpu.make_async_copy(
                    gb[b], acc.at[cols_v.at[b]], sem_s[b]).wait()

        pltpu.sync_copy(u_hbm.at[pl.ds(s * nrs, nrs), pl.ds(c * hd, hd)],
                        u_sh.at[pl.ds(s * nrs, nrs)])
        zero_gb0()
        zero_acc_slice()
        pltpu.sync_copy(row_hbm.at[pl.ds(s * nch2, nch2)], rows_v)
        pltpu.sync_copy(col_hbm.at[pl.ds(s * nch2, nch2)], cols_v)

        plsc.subcore_barrier()
        prop_loop()
        plsc.subcore_barrier()

        def p2chunk(q, _):
            base = s * nrs + q * pch
            pltpu.sync_copy(acc.at[pl.ds(base, pch)], gvb)
            pltpu.sync_copy(a_hbm.at[pl.ds(base, pch), pl.ds(c * hd, hd)],
                            avb)
            pltpu.sync_copy(dinv_hbm.at[pl.ds(base, pch)], dvb)

            def scale(r, __):
                dv = dvb[r, pl.ds(0, L)]
                for k in range(hd // L):
                    g = gvb[r, pl.ds(k * L, L)]
                    a = avb[r, pl.ds(k * L, L)]
                    avb[r, pl.ds(k * L, L)] = dv * (a - 2.0 * dv * g)
                return 0
            lax.fori_loop(0, pch, scale, 0)
            pltpu.sync_copy(avb, u_sh.at[pl.ds(base, pch)])
            return 0
        lax.fori_loop(0, nrs // pch, p2chunk, 0)
        plsc.subcore_barrier()
        zero_gb0()
        zero_acc_slice()
        plsc.subcore_barrier()

        prop_loop()
        plsc.subcore_barrier()
        pltpu.sync_copy(acc.at[pl.ds(s * R, R)],
                        out_hbm.at[pl.ds(s * R, R), pl.ds(c * hd, hd)])

    return layer


def _row_spec(d):
    return pl.BlockSpec((BLK, d), lambda i: (i, 0))


def _parts_spec(d):
    return pl.BlockSpec((NC, BLK, d), lambda i: (0, i, 0))


def _full_spec(shape):
    nd = len(shape)
    return pl.BlockSpec(shape, lambda i, _nd=nd: (0,) * _nd)


def _tc1a(x, w1):
    def body(x_ref, w_ref, xw0_ref, xw1_ref, xw2_ref):
        xb = x_ref[...]
        w = w_ref[...]
        xw0_ref[...] = jnp.dot(xb, w[0] - w[2],
                               preferred_element_type=jnp.float32)
        xw1_ref[...] = jnp.dot(xb, w[1], preferred_element_type=jnp.float32)
        xw2_ref[...] = jnp.dot(xb, w[2], preferred_element_type=jnp.float32)

    o = jax.ShapeDtypeStruct((N, HID), jnp.float32)
    return pl.pallas_call(
        body,
        grid=(GRID,),
        in_specs=[_row_spec(F_IN), _full_spec((3, F_IN, HID))],
        out_specs=[_row_spec(HID), _row_spec(HID), _row_spec(HID)],
        out_shape=[o, o, o],
    )(x, w1)


def _tc1b(xw2, parts):
    def body(xw2_ref, p_ref, u1_ref, dinv_ref):
        degb = p_ref[0] + p_ref[1]
        dinv = jnp.where(degb > 0,
                         lax.rsqrt(jnp.where(degb > 0, degb, 1.0)), 0.0)
        u1_ref[...] = dinv * xw2_ref[...]
        dinv_ref[...] = jnp.broadcast_to(dinv, (BLK, 16))

    return pl.pallas_call(
        body,
        grid=(GRID,),
        in_specs=[_row_spec(HID), _parts_spec(1)],
        out_specs=[_row_spec(HID), _row_spec(16)],
        out_shape=[jax.ShapeDtypeStruct((N, HID), jnp.float32),
                   jax.ShapeDtypeStruct((N, 16), jnp.float32)],
    )(xw2, parts)


def _tc3(xw0, parts, dinv, b1, w2):
    def body(xw0_ref, p_ref, dinv_ref, b_ref, w_ref,
             y0_ref, y1_ref, u2_ref):
        g = p_ref[...]
        dinv = dinv_ref[:, :1]
        h = jax.nn.relu(xw0_ref[...] - dinv * g + b_ref[...])
        w = w_ref[...]
        y0_ref[...] = jnp.dot(h, w[0] - w[2],
                              preferred_element_type=jnp.float32)
        y1_ref[...] = jnp.dot(h, w[1], preferred_element_type=jnp.float32)
        u2_ref[...] = dinv * jnp.dot(h, w[2],
                                     preferred_element_type=jnp.float32)

    o = jax.ShapeDtypeStruct((N, CLS), jnp.float32)
    return pl.pallas_call(
        body,
        grid=(GRID,),
        in_specs=[_row_spec(HID), _row_spec(HID), _row_spec(16),
                  _full_spec((1, HID)), _full_spec((3, HID, CLS))],
        out_specs=[_row_spec(CLS), _row_spec(CLS), _row_spec(CLS)],
        out_shape=[o, o, o],
    )(xw0, parts, dinv, b1, w2)


def _tc5(y0, parts, dinv, b2):
    def body(y0_ref, p_ref, dinv_ref, b_ref, out_ref):
        z = y0_ref[...] - dinv_ref[:, :1] * p_ref[...] + b_ref[...]
        m = jnp.max(z, axis=1, keepdims=True)
        e = jnp.exp(z - m)
        out_ref[...] = (z - m) - jnp.log(jnp.sum(e, axis=1, keepdims=True))

    return pl.pallas_call(
        body,
        grid=(GRID,),
        in_specs=[_row_spec(CLS), _row_spec(CLS), _row_spec(16),
                  _full_spec((1, CLS))],
        out_specs=_row_spec(CLS),
        out_shape=jax.ShapeDtypeStruct((N, CLS), jnp.float32),
    )(y0, parts, dinv, b2)


def kernel(x, edge_index, W1, b1, W2, b2):
    e = edge_index.shape[1]
    nch2 = -(-e // (NS * CH))
    nch2 = -(-nch2 // NBUF) * NBUF
    e_pad = NS * CH * nch2
    row = jnp.pad(edge_index[0], (0, e_pad - e)).reshape(e_pad // CH, CH)
    col = jnp.pad(edge_index[1], (0, e_pad - e)).reshape(e_pad // CH, CH)

    xw0, xw1, xw2 = _tc1a(x, W1)
    degp2, _, colf = _make_deg(nch2 // NC)(row, col)
    degp = degp2.reshape(NC, N_ACC, 1)
    u1, dinv = _tc1b(xw2, degp)

    g2 = _make_layer(HID, nch2)(u1, xw1, dinv, row, colf)
    y0, y1, u2 = _tc3(xw0, g2, dinv, b1.reshape(1, HID), W2)
    g4 = _make_layer(CLS, nch2)(u2, y1, dinv, row, colf)
    return _tc5(y0, g4, dinv, b2.reshape(1, CLS))
